# baseline (speedup 1.0000x reference)
import functools
import math

import jax
import jax.numpy as jnp
import numpy as np
from jax import lax
from jax.experimental import pallas as pl
from jax.experimental.pallas import tpu as pltpu

NB = 16
CUTOFF = 6.0
NITER = 2
NRES = 3
F = 32
NZP = 128

_BINOM = np.array([math.comb(NB - 1, k) for k in range(NB)], dtype=np.float32)

INTERPRET = False


def _silu(x):
    return x * jax.nn.sigmoid(x)


def _dsilu(x):
    s = jax.nn.sigmoid(x)
    return s * (1.0 + x * (1.0 - s))


def _basis_from_disp(disp, binom):
    d2 = jnp.sum(disp * disp, axis=-1, keepdims=True) + 1e-12
    d = jnp.sqrt(d2)
    u = d / (1.0 + d)
    du_dd = 1.0 / ((1.0 + d) * (1.0 + d))
    ks = lax.broadcasted_iota(jnp.int32, (1, NB), 1).astype(jnp.float32)
    n = float(NB - 1)
    logu = jnp.log(u)
    log1mu = jnp.log(1.0 - u)
    up = jnp.exp(ks * logu)
    um = jnp.exp((n - ks) * log1mu)
    bern = binom * up * um
    upm1 = jnp.exp(jnp.maximum(ks - 1.0, 0.0) * logu)
    umm1 = jnp.exp(jnp.maximum(n - ks - 1.0, 0.0) * log1mu)
    dbern_du = binom * (ks * upm1 * um - (n - ks) * up * umm1)
    x = d / CUTOFF
    inb = x < 1.0
    den = jnp.where(inb, 1.0 - x * x, 1.0)
    cut = jnp.where(inb, jnp.exp(1.0 - 1.0 / den), 0.0)
    dcut_dd = jnp.where(inb, cut * (-2.0 * x / (den * den)) * (1.0 / CUTOFF), 0.0)
    basis = bern * cut
    dbasis = dbern_du * (du_dd * cut) + bern * dcut_dd
    return d, basis, dbasis


EBLK = 2000
NBLK = 2000


def _full_spec(shape):
    nd = len(shape)
    return pl.BlockSpec(shape, lambda i: (0,) * nd)


def _row_spec(blk, ncol):
    return pl.BlockSpec((blk, ncol), lambda i: (i, 0))


def _tc_call(body, grid, in_specs, out_specs, out_shape):
    return pl.pallas_call(
        body, grid=(grid,), in_specs=in_specs, out_specs=out_specs,
        out_shape=out_shape, interpret=INTERPRET)


def _embed_body(z_ref, embp_ref, ebp_ref, x0_ref, eb_ref):
    z = z_ref[...]
    cols = lax.broadcasted_iota(jnp.int32, (z.shape[0], NZP), 1)
    onehot = jnp.where(cols == z, 1.0, 0.0).astype(jnp.float32)
    x0_ref[...] = jnp.dot(onehot, embp_ref[...], preferred_element_type=jnp.float32,
                          precision=lax.Precision.HIGHEST)
    eb_ref[...] = jnp.dot(onehot, ebp_ref[...], preferred_element_type=jnp.float32,
                          precision=lax.Precision.HIGHEST)


def tc_embed(z, embp, ebp):
    n = z.shape[0]
    return _tc_call(
        _embed_body, n // NBLK,
        [_row_spec(NBLK, 1), _full_spec((NZP, F)), _full_spec((NZP, 1))],
        [_row_spec(NBLK, F), _row_spec(NBLK, 1)],
        [jax.ShapeDtypeStruct((n, F), jnp.float32),
         jax.ShapeDtypeStruct((n, 1), jnp.float32)],
    )(z, embp, ebp)


def _basis_body(disp_ref, binom_ref, basis_ref):
    _, basis, _ = _basis_from_disp(disp_ref[...], binom_ref[...])
    basis_ref[...] = basis


def tc_basis(disp, binom):
    e = disp.shape[0]
    return _tc_call(
        _basis_body, e // EBLK,
        [_row_spec(EBLK, 16), _full_spec((1, NB))],
        _row_spec(EBLK, NB),
        jax.ShapeDtypeStruct((e, NB), jnp.float32),
    )(disp, binom)


def _msg_body(xsrc_ref, basis_ref, wb_ref, msg_ref):
    radial = jnp.dot(basis_ref[...], wb_ref[...], preferred_element_type=jnp.float32)
    msg_ref[...] = xsrc_ref[...] * radial


def tc_msg(xsrc, basis, wb):
    e = xsrc.shape[0]
    return _tc_call(
        _msg_body, e // EBLK,
        [_row_spec(EBLK, F), _row_spec(EBLK, NB), _full_spec((NB, F))],
        _row_spec(EBLK, F),
        jax.ShapeDtypeStruct((e, F), jnp.float32),
    )(xsrc, basis, wb)


def _mlp_fwd_math(a, w1, b1, w2, b2, w3, b3):
    As, Hs = [], []
    q = a
    for r in range(NRES):
        As.append(a)
        s = _silu(a)
        h = jnp.dot(s, w1[r], preferred_element_type=jnp.float32) + b1[r]
        Hs.append(h)
        t = jnp.maximum(h, 0.0)
        q = jnp.dot(t, w2[r], preferred_element_type=jnp.float32) + b2[r]
        a = a + q
    v = jnp.dot(q, w3, preferred_element_type=jnp.float32) + b3
    return a + _silu(v), (As, Hs, v)


def _mlp_body(u_ref, w1_ref, b1_ref, w2_ref, b2_ref, w3_ref, b3_ref, x_ref):
    b1 = b1_ref[...]
    b2 = b2_ref[...]
    x, _ = _mlp_fwd_math(
        u_ref[...], w1_ref[...], [b1[r:r + 1] for r in range(NRES)],
        w2_ref[...], [b2[r:r + 1] for r in range(NRES)],
        w3_ref[...], b3_ref[...])
    x_ref[...] = x


def tc_mlp(u, w1, b1, w2, b2, w3, b3):
    n = u.shape[0]
    return _tc_call(
        _mlp_body, n // NBLK,
        [_row_spec(NBLK, F), _full_spec((NRES, F, F)), _full_spec((NRES, F)),
         _full_spec((NRES, F, F)), _full_spec((NRES, F)),
         _full_spec((F, F)), _full_spec((1, F))],
        _row_spec(NBLK, F),
        jax.ShapeDtypeStruct((n, F), jnp.float32),
    )(u, w1, b1, w2, b2, w3, b3)


def _mlp_bwd_body(u_ref, g_ref, w1_ref, b1_ref, w2_ref, b2_ref, w3_ref, b3_ref,
                  w1t_ref, w2t_ref, w3t_ref, gu_ref):
    b1 = b1_ref[...]
    b2 = b2_ref[...]
    _, (As, Hs, v) = _mlp_fwd_math(
        u_ref[...], w1_ref[...], [b1[r:r + 1] for r in range(NRES)],
        w2_ref[...], [b2[r:r + 1] for r in range(NRES)],
        w3_ref[...], b3_ref[...])
    g_out = g_ref[...]
    g_a = g_out
    g_q_extra = jnp.dot(g_out * _dsilu(v), w3t_ref[...],
                        preferred_element_type=jnp.float32)
    w1t = w1t_ref[...]
    w2t = w2t_ref[...]
    for r in range(NRES - 1, -1, -1):
        g_q = g_a + g_q_extra if r == NRES - 1 else g_a
        g_t = jnp.dot(g_q, w2t[r], preferred_element_type=jnp.float32)
        g_h = jnp.where(Hs[r] > 0.0, g_t, 0.0)
        g_s = jnp.dot(g_h, w1t[r], preferred_element_type=jnp.float32)
        g_a = g_a + g_s * _dsilu(As[r])
    gu_ref[...] = g_a


def tc_mlp_bwd(u, g_out, w1, b1, w2, b2, w3, b3, w1t, w2t, w3t):
    n = u.shape[0]
    return _tc_call(
        _mlp_bwd_body, n // NBLK,
        [_row_spec(NBLK, F), _row_spec(NBLK, F),
         _full_spec((NRES, F, F)), _full_spec((NRES, F)),
         _full_spec((NRES, F, F)), _full_spec((NRES, F)),
         _full_spec((F, F)), _full_spec((1, F)),
         _full_spec((NRES, F, F)), _full_spec((NRES, F, F)), _full_spec((F, F))],
        _row_spec(NBLK, F),
        jax.ShapeDtypeStruct((n, F), jnp.float32),
    )(u, g_out, w1, b1, w2, b2, w3, b3, w1t, w2t, w3t)


def _head_body(x2_ref, eb_ref, am_ref, we1_ref, we1t_ref, we2_ref,
               ae16_ref, gx2_ref):
    x2 = x2_ref[...]
    am = am_ref[...]
    t = jnp.dot(x2, we1_ref[...], preferred_element_type=jnp.float32)
    we2 = we2_ref[...]
    ae = _silu(t) * we2 + eb_ref[...]
    ae = ae * am
    col0 = lax.broadcasted_iota(jnp.int32, (1, 16), 1) == 0
    ae16_ref[...] = jnp.where(col0, ae, 0.0)
    g_t = (-am) * we2 * _dsilu(t)
    gx2_ref[...] = g_t * we1t_ref[...]


def tc_head(x2, eb, am, we1, we1t, we2):
    n = x2.shape[0]
    return _tc_call(
        _head_body, n // NBLK,
        [_row_spec(NBLK, F), _row_spec(NBLK, 1), _row_spec(NBLK, 1),
         _full_spec((F, 1)), _full_spec((1, F)), _full_spec((1, 1))],
        [_row_spec(NBLK, 16), _row_spec(NBLK, F)],
        [jax.ShapeDtypeStruct((n, 16), jnp.float32),
         jax.ShapeDtypeStruct((n, F), jnp.float32)],
    )(x2, eb, am, we1, we1t, we2)


def _edge_bwd_body(with_gscat, gudst_ref, xsrc_ref, basis_ref, wb_ref, wbt_ref,
                   gbasis_ref, *rest):
    gudst = gudst_ref[...]
    gbasis_ref[...] = jnp.dot(gudst * xsrc_ref[...], wbt_ref[...],
                              preferred_element_type=jnp.float32)
    if with_gscat:
        radial = jnp.dot(basis_ref[...], wb_ref[...],
                         preferred_element_type=jnp.float32)
        rest[0][...] = gudst * radial


def tc_edge_bwd(gudst, xsrc, basis, wb, wbt, with_gscat):
    e = gudst.shape[0]
    out_specs = [_row_spec(EBLK, NB)]
    out_shape = [jax.ShapeDtypeStruct((e, NB), jnp.float32)]
    if with_gscat:
        out_specs.append(_row_spec(EBLK, F))
        out_shape.append(jax.ShapeDtypeStruct((e, F), jnp.float32))
    return _tc_call(
        functools.partial(_edge_bwd_body, with_gscat), e // EBLK,
        [_row_spec(EBLK, F), _row_spec(EBLK, F), _row_spec(EBLK, NB),
         _full_spec((NB, F)), _full_spec((F, NB))],
        out_specs, out_shape,
    )(gudst, xsrc, basis, wb, wbt)


def _geom_bwd_body(disp_ref, gb0_ref, gb1_ref, binom_ref, gdisp_ref):
    disp = disp_ref[...]
    d, _, dbasis = _basis_from_disp(disp, binom_ref[...])
    gb = gb0_ref[...] + gb1_ref[...]
    g_d = jnp.sum(gb * dbasis, axis=-1, keepdims=True)
    gdisp_ref[...] = (g_d / d) * disp


def tc_geom_bwd(disp, gb0, gb1, binom):
    e = disp.shape[0]
    return _tc_call(
        _geom_bwd_body, e // EBLK,
        [_row_spec(EBLK, 16), _row_spec(EBLK, NB), _row_spec(EBLK, NB),
         _full_spec((1, NB))],
        _row_spec(EBLK, 16),
        jax.ShapeDtypeStruct((e, 16), jnp.float32),
    )(disp, gb0, gb1, binom)


def _finalize_body(s_ref, am_ref, f_ref):
    s = s_ref[...]
    f_ref[...] = (s[:, :16] - s[:, 16:]) * am_ref[...]


def tc_finalize(s, am):
    n = s.shape[0]
    return _tc_call(
        _finalize_body, n // NBLK,
        [_row_spec(NBLK, 32), _row_spec(NBLK, 1)],
        _row_spec(NBLK, 16),
        jax.ShapeDtypeStruct((n, 16), jnp.float32),
    )(s, am)


def gather_rows(table, idx):
    return jnp.take(table, idx, axis=0)


def gather_sub(pos16, src, dst):
    return jnp.take(pos16, src, axis=0) - jnp.take(pos16, dst, axis=0)


def scatter_add_rows(vals, idx, nrows):
    return jax.ops.segment_sum(vals, idx, num_segments=nrows)


def scatter_dual16(vals, src, dst, nrows):
    a = jax.ops.segment_sum(vals, src, num_segments=nrows)
    b = jax.ops.segment_sum(vals, dst, num_segments=nrows)
    return jnp.concatenate([a, b], axis=1)


def scatter_energy(ae16, segs, nseg):
    return jax.ops.segment_sum(ae16, segs, num_segments=nseg)


def kernel(atomic_numbers, positions, dst_idx, src_idx, batch_segments,
           batch_size, batch_mask, atom_mask, emb, Wb, W1, b1, W2, b2, W3, b3,
           We1, We2, element_bias):
    n = positions.shape[0]
    e = dst_idx.shape[0]
    nb_batches = 1000

    z = atomic_numbers.astype(jnp.int32).reshape(n, 1)
    src = src_idx.astype(jnp.int32)
    dst = dst_idx.astype(jnp.int32)
    segs = batch_segments.astype(jnp.int32)
    am = atom_mask.reshape(n, 1)
    pos16 = jnp.pad(positions, ((0, 0), (0, 13)))
    embp = jnp.pad(emb, ((0, NZP - emb.shape[0]), (0, 0)))
    ebp = jnp.pad(element_bias.reshape(-1, 1), ((0, NZP - element_bias.shape[0]), (0, 0)))
    w1t = jnp.swapaxes(W1, -1, -2)
    w2t = jnp.swapaxes(W2, -1, -2)
    w3t = jnp.swapaxes(W3, -1, -2)
    wbt = jnp.swapaxes(Wb, -1, -2)
    we1t = We1.reshape(1, F)
    b3r = b3.reshape(NITER, 1, F)
    binom = jnp.asarray(_BINOM).reshape(1, NB)

    disp = gather_sub(pos16, src, dst)
    basis = tc_basis(disp, binom)
    x0, eb = tc_embed(z, embp, ebp)

    xs = [x0]
    us = []
    xsrcs = []
    x = x0
    for i in range(NITER):
        xsrc = gather_rows(x, src)
        xsrcs.append(xsrc)
        msg = tc_msg(xsrc, basis, Wb[i])
        u = scatter_add_rows(msg, dst, n)
        us.append(u)
        x = tc_mlp(u, W1[i], b1[i], W2[i], b2[i], W3[i], b3r[i])
        xs.append(x)

    ae16, g_x = tc_head(x, eb, am, We1, we1t, We2)
    energy16 = scatter_energy(ae16, segs, nb_batches)
    energy = energy16[:, 0]

    gbs = [None, None]
    for i in range(NITER - 1, -1, -1):
        g_u = tc_mlp_bwd(us[i], g_x, W1[i], b1[i], W2[i], b2[i], W3[i], b3r[i],
                         w1t[i], w2t[i], w3t[i])
        gudst = gather_rows(g_u, dst)
        if i > 0:
            gbs[i], gscat = tc_edge_bwd(gudst, xsrcs[i], basis, Wb[i], wbt[i], True)
            g_x = scatter_add_rows(gscat, src, n)
        else:
            (gbs[i],) = tc_edge_bwd(gudst, xsrcs[i], basis, Wb[i], wbt[i], False)

    gdisp = tc_geom_bwd(disp, gbs[0], gbs[1], binom)
    s = scatter_dual16(gdisp, src, dst, n)
    forces16 = tc_finalize(s, am)
    forces = forces16[:, :3]
    return energy, forces


# speedup vs baseline: 3.2474x; 3.2474x over previous
import functools
import math

import jax
import jax.numpy as jnp
import numpy as np
from jax import lax
from jax.experimental import pallas as pl
from jax.experimental.pallas import tpu as pltpu
from jax.experimental.pallas import tpu_sc as plsc

NB = 16
CUTOFF = 6.0
NITER = 2
NRES = 3
F = 32
NZP = 128

_BINOM = np.array([math.comb(NB - 1, k) for k in range(NB)], dtype=np.float32)


def _silu(x):
    return x * jax.nn.sigmoid(x)


def _dsilu(x):
    s = jax.nn.sigmoid(x)
    return s * (1.0 + x * (1.0 - s))


def _basis_math(d, binom128):
    u = d / (1.0 + d)
    du_dd = 1.0 / ((1.0 + d) * (1.0 + d))
    ks = jnp.remainder(
        lax.broadcasted_iota(jnp.int32, (1, 128), 1), 32).astype(jnp.float32)
    n = float(NB - 1)
    logu = jnp.log(u)
    log1mu = jnp.log(1.0 - u)
    up = jnp.exp(ks * logu)
    um = jnp.exp((n - ks) * log1mu)
    bern = binom128 * up * um
    upm1 = jnp.exp(jnp.maximum(ks - 1.0, 0.0) * logu)
    umm1 = jnp.exp(jnp.maximum(n - ks - 1.0, 0.0) * log1mu)
    dbern_du = binom128 * (ks * upm1 * um - (n - ks) * up * umm1)
    x = d / CUTOFF
    inb = x < 1.0
    den = jnp.where(inb, 1.0 - x * x, 1.0)
    cut = jnp.where(inb, jnp.exp(1.0 - 1.0 / den), 0.0)
    dcut_dd = jnp.where(inb, cut * (-2.0 * x / (den * den)) * (1.0 / CUTOFF), 0.0)
    basis = bern * cut
    dbasis = dbern_du * (du_dd * cut) + bern * dcut_dd
    return basis, dbasis


EBLK = 1000
NBLK = 2000


def _full_spec(shape):
    nd = len(shape)
    return pl.BlockSpec(shape, lambda i: (0,) * nd)


def _row_spec(blk, ncol):
    return pl.BlockSpec((blk, ncol), lambda i: (i, 0))


def _tc_call(body, grid, in_specs, out_specs, out_shape):
    return pl.pallas_call(body, grid=(grid,), in_specs=in_specs,
                          out_specs=out_specs, out_shape=out_shape)


def _embed_body(z_ref, embp_ref, ebp_ref, x0_ref, eb_ref):
    z = z_ref[...]
    cols = lax.broadcasted_iota(jnp.int32, (z.shape[0], NZP), 1)
    onehot = jnp.where(cols == z, 1.0, 0.0).astype(jnp.float32)
    x0_ref[...] = jnp.dot(onehot, embp_ref[...], preferred_element_type=jnp.float32,
                          precision=lax.Precision.HIGHEST)
    eb_ref[...] = jnp.dot(onehot, ebp_ref[...], preferred_element_type=jnp.float32,
                          precision=lax.Precision.HIGHEST)


def tc_embed(z, embp, ebp):
    n = z.shape[0]
    return _tc_call(
        _embed_body, n // NBLK,
        [_row_spec(NBLK, 1), _full_spec((NZP, F)), _full_spec((NZP, 1))],
        [_row_spec(NBLK, F), _row_spec(NBLK, 1)],
        [jax.ShapeDtypeStruct((n, F), jnp.float32),
         jax.ShapeDtypeStruct((n, 1), jnp.float32)],
    )(z, embp, ebp)


def _basis_body(psrc_ref, pdst_ref, bdones_ref, binom_ref, disp_ref, basis_ref):
    disp = psrc_ref[...] - pdst_ref[...]
    disp_ref[...] = disp
    d2 = jnp.dot(disp * disp, bdones_ref[...], preferred_element_type=jnp.float32,
                 precision=lax.Precision.HIGHEST) + 1e-12
    d = jnp.sqrt(d2)
    basis, _ = _basis_math(d, binom_ref[...])
    basis_ref[...] = basis


def tc_basis(psrc_p, pdst_p, bdones, binom128):
    r = psrc_p.shape[0]
    return _tc_call(
        _basis_body, r // EBLK,
        [_row_spec(EBLK, 128), _row_spec(EBLK, 128),
         _full_spec((128, 128)), _full_spec((1, 128))],
        [_row_spec(EBLK, 128), _row_spec(EBLK, 128)],
        [jax.ShapeDtypeStruct((r, 128), jnp.float32),
         jax.ShapeDtypeStruct((r, 128), jnp.float32)],
    )(psrc_p, pdst_p, bdones, binom128)


def _msg_body(xsrc_ref, basis_ref, bdwb_ref, msg_ref):
    radial = jnp.dot(basis_ref[...], bdwb_ref[...],
                     preferred_element_type=jnp.float32)
    msg_ref[...] = xsrc_ref[...] * radial


def tc_msg(xsrc_p, basis_p, bdwb):
    r = xsrc_p.shape[0]
    return _tc_call(
        _msg_body, r // EBLK,
        [_row_spec(EBLK, 128), _row_spec(EBLK, 128), _full_spec((128, 128))],
        _row_spec(EBLK, 128),
        jax.ShapeDtypeStruct((r, 128), jnp.float32),
    )(xsrc_p, basis_p, bdwb)


def _mlp_fwd_math(a, w1, b1, w2, b2, w3, b3):
    As, Hs = [], []
    q = a
    for r in range(NRES):
        As.append(a)
        s = _silu(a)
        h = jnp.dot(s, w1[r], preferred_element_type=jnp.float32) + b1[r]
        Hs.append(h)
        t = jnp.maximum(h, 0.0)
        q = jnp.dot(t, w2[r], preferred_element_type=jnp.float32) + b2[r]
        a = a + q
    v = jnp.dot(q, w3, preferred_element_type=jnp.float32) + b3
    return a + _silu(v), (As, Hs, v)


def _mlp_body(u_ref, w1_ref, b1_ref, w2_ref, b2_ref, w3_ref, b3_ref, x_ref):
    b1 = b1_ref[...]
    b2 = b2_ref[...]
    x, _ = _mlp_fwd_math(
        u_ref[...], w1_ref[...], [b1[r:r + 1] for r in range(NRES)],
        w2_ref[...], [b2[r:r + 1] for r in range(NRES)],
        w3_ref[...], b3_ref[...])
    x_ref[...] = x


def tc_mlp(u, w1, b1, w2, b2, w3, b3):
    n = u.shape[0]
    return _tc_call(
        _mlp_body, n // NBLK,
        [_row_spec(NBLK, F), _full_spec((NRES, F, F)), _full_spec((NRES, F)),
         _full_spec((NRES, F, F)), _full_spec((NRES, F)),
         _full_spec((F, F)), _full_spec((1, F))],
        _row_spec(NBLK, F),
        jax.ShapeDtypeStruct((n, F), jnp.float32),
    )(u, w1, b1, w2, b2, w3, b3)


def _mlp_bwd_body(u_ref, g_ref, w1_ref, b1_ref, w2_ref, b2_ref, w3_ref, b3_ref,
                  w1t_ref, w2t_ref, w3t_ref, gu_ref):
    b1 = b1_ref[...]
    b2 = b2_ref[...]
    _, (As, Hs, v) = _mlp_fwd_math(
        u_ref[...], w1_ref[...], [b1[r:r + 1] for r in range(NRES)],
        w2_ref[...], [b2[r:r + 1] for r in range(NRES)],
        w3_ref[...], b3_ref[...])
    g_out = g_ref[...]
    g_a = g_out
    g_q_extra = jnp.dot(g_out * _dsilu(v), w3t_ref[...],
                        preferred_element_type=jnp.float32)
    w1t = w1t_ref[...]
    w2t = w2t_ref[...]
    for r in range(NRES - 1, -1, -1):
        g_q = g_a + g_q_extra if r == NRES - 1 else g_a
        g_t = jnp.dot(g_q, w2t[r], preferred_element_type=jnp.float32)
        g_h = jnp.where(Hs[r] > 0.0, g_t, 0.0)
        g_s = jnp.dot(g_h, w1t[r], preferred_element_type=jnp.float32)
        g_a = g_a + g_s * _dsilu(As[r])
    gu_ref[...] = g_a


def tc_mlp_bwd(u, g_out, w1, b1, w2, b2, w3, b3, w1t, w2t, w3t):
    n = u.shape[0]
    return _tc_call(
        _mlp_bwd_body, n // NBLK,
        [_row_spec(NBLK, F), _row_spec(NBLK, F),
         _full_spec((NRES, F, F)), _full_spec((NRES, F)),
         _full_spec((NRES, F, F)), _full_spec((NRES, F)),
         _full_spec((F, F)), _full_spec((1, F)),
         _full_spec((NRES, F, F)), _full_spec((NRES, F, F)), _full_spec((F, F))],
        _row_spec(NBLK, F),
        jax.ShapeDtypeStruct((n, F), jnp.float32),
    )(u, g_out, w1, b1, w2, b2, w3, b3, w1t, w2t, w3t)


def _head_body(x2_ref, eb_ref, am_ref, we1_ref, we1t_ref, we2_ref,
               ae_ref, gx2_ref):
    x2 = x2_ref[...]
    am = am_ref[...]
    t = jnp.dot(x2, we1_ref[...], preferred_element_type=jnp.float32)
    we2 = we2_ref[...]
    ae_ref[...] = (_silu(t) * we2 + eb_ref[...]) * am
    g_t = (-am) * we2 * _dsilu(t)
    gx2_ref[...] = g_t * we1t_ref[...]


def tc_head(x2, eb, am, we1, we1t, we2):
    n = x2.shape[0]
    return _tc_call(
        _head_body, n // NBLK,
        [_row_spec(NBLK, F), _row_spec(NBLK, 1), _row_spec(NBLK, 1),
         _full_spec((F, 1)), _full_spec((1, F)), _full_spec((1, 1))],
        [_row_spec(NBLK, 1), _row_spec(NBLK, F)],
        [jax.ShapeDtypeStruct((n, 1), jnp.float32),
         jax.ShapeDtypeStruct((n, F), jnp.float32)],
    )(x2, eb, am, we1, we1t, we2)


def _energy_body(ae_ref, segs_ref, out_ref):
    @pl.when(pl.program_id(0) == 0)
    def _():
        out_ref[...] = jnp.zeros_like(out_ref)
    segs = segs_ref[...]
    cols = lax.broadcasted_iota(jnp.int32, (segs.shape[0], 1024), 1)
    onehot = jnp.where(cols == segs, 1.0, 0.0).astype(jnp.float32)
    out_ref[...] += lax.dot_general(
        onehot, ae_ref[...], (((0,), (0,)), ((), ())),
        preferred_element_type=jnp.float32, precision=lax.Precision.HIGHEST)


def tc_energy(ae, segs):
    n = ae.shape[0]
    return _tc_call(
        _energy_body, n // NBLK,
        [_row_spec(NBLK, 1), _row_spec(NBLK, 1)],
        _full_spec((1024, 1)),
        jax.ShapeDtypeStruct((1024, 1), jnp.float32),
    )(ae, segs)


def _edge_bwd_body(with_gscat, gudst_ref, xsrc_ref, basis_ref, bdwb_ref,
                   bdwbt_ref, gbasis_ref, *rest):
    gudst = gudst_ref[...]
    gbasis_ref[...] = jnp.dot(gudst * xsrc_ref[...], bdwbt_ref[...],
                              preferred_element_type=jnp.float32)
    if with_gscat:
        radial = jnp.dot(basis_ref[...], bdwb_ref[...],
                         preferred_element_type=jnp.float32)
        rest[0][...] = gudst * radial


def tc_edge_bwd(gudst_p, xsrc_p, basis_p, bdwb, bdwbt, with_gscat):
    r = gudst_p.shape[0]
    out_specs = [_row_spec(EBLK, 128)]
    out_shape = [jax.ShapeDtypeStruct((r, 128), jnp.float32)]
    if with_gscat:
        out_specs.append(_row_spec(EBLK, 128))
        out_shape.append(jax.ShapeDtypeStruct((r, 128), jnp.float32))
    return _tc_call(
        functools.partial(_edge_bwd_body, with_gscat), r // EBLK,
        [_row_spec(EBLK, 128), _row_spec(EBLK, 128), _row_spec(EBLK, 128),
         _full_spec((128, 128)), _full_spec((128, 128))],
        out_specs, out_shape,
    )(gudst_p, xsrc_p, basis_p, bdwb, bdwbt)


def _geom_bwd_body(disp_ref, gb0_ref, gb1_ref, bdones_ref, binom_ref, gdisp_ref):
    disp = disp_ref[...]
    d2 = jnp.dot(disp * disp, bdones_ref[...], preferred_element_type=jnp.float32,
                 precision=lax.Precision.HIGHEST) + 1e-12
    d = jnp.sqrt(d2)
    _, dbasis = _basis_math(d, binom_ref[...])
    gb = gb0_ref[...] + gb1_ref[...]
    g_d = jnp.dot(gb * dbasis, bdones_ref[...], preferred_element_type=jnp.float32,
                  precision=lax.Precision.HIGHEST)
    gdisp_ref[...] = (g_d / d) * disp


def tc_geom_bwd(disp_p, gb0_p, gb1_p, bdones, binom128):
    r = disp_p.shape[0]
    return _tc_call(
        _geom_bwd_body, r // EBLK,
        [_row_spec(EBLK, 128), _row_spec(EBLK, 128), _row_spec(EBLK, 128),
         _full_spec((128, 128)), _full_spec((1, 128))],
        _row_spec(EBLK, 128),
        jax.ShapeDtypeStruct((r, 128), jnp.float32),
    )(disp_p, gb0_p, gb1_p, bdones, binom128)


def _finalize_body(ssrc_ref, sdst_ref, am_ref, f_ref):
    f_ref[...] = (ssrc_ref[...] - sdst_ref[...]) * am_ref[...]


def tc_finalize(ssrc, sdst, am):
    n = ssrc.shape[0]
    return _tc_call(
        _finalize_body, n // NBLK,
        [_row_spec(NBLK, F), _row_spec(NBLK, F), _row_spec(NBLK, 1)],
        _row_spec(NBLK, F),
        jax.ShapeDtypeStruct((n, F), jnp.float32),
    )(ssrc, sdst, am)


NCORES = 2
NSUB = 16
NW = NCORES * NSUB
CH = 80
HALF = 50000
STRIPE = 3128
NACC = NSUB * STRIPE
DUMP = 50040


def sc_gather(table, idx):
    e = idx.shape[0]
    per_w = e // NW
    nch = per_w // CH
    mesh = plsc.VectorSubcoreMesh(core_axis_name="c", subcore_axis_name="s")

    @functools.partial(
        pl.kernel, mesh=mesh,
        compiler_params=pltpu.CompilerParams(use_tc_tiling_on_sc=False),
        out_type=jax.ShapeDtypeStruct((e, F), jnp.float32),
        scratch_types=[
            pltpu.VMEM((2, CH), jnp.int32),
            pltpu.VMEM((2, CH, F), jnp.float32),
            pltpu.SemaphoreType.DMA,
            pltpu.SemaphoreType.DMA,
        ],
    )
    def k(table_h, idx_h, out_h, idxb, rowb, sem0, sem1):
        wid = lax.axis_index("s") * NCORES + lax.axis_index("c")
        base = wid * per_w
        sems = (sem0, sem1)

        def start(ch, buf):
            pltpu.sync_copy(idx_h.at[pl.ds(base + ch * CH, CH)], idxb.at[buf])
            pltpu.async_copy(table_h.at[idxb.at[buf]], rowb.at[buf], sems[buf])

        def fin(ch, buf):
            pltpu.make_async_copy(table_h.at[idxb.at[buf]], rowb.at[buf],
                                  sems[buf]).wait()
            pltpu.sync_copy(rowb.at[buf],
                            out_h.at[pl.ds(base + ch * CH, CH), :])

        start(0, 0)

        def body(ch, carry):
            @pl.when(lax.rem(ch, 2) == 0)
            def _():
                @pl.when(ch + 1 < nch)
                def _():
                    start(ch + 1, 1)
                fin(ch, 0)

            @pl.when(lax.rem(ch, 2) == 1)
            def _():
                @pl.when(ch + 1 < nch)
                def _():
                    start(ch + 1, 0)
                fin(ch, 1)

            return carry

        lax.fori_loop(0, nch, body, 0)

    return k(table, idx)


def sc_scatter_add(vals, idx, zinit):
    e = idx.shape[0]
    per_w = e // NSUB
    nch = per_w // CH
    mesh = plsc.VectorSubcoreMesh(core_axis_name="c", subcore_axis_name="s")

    @functools.partial(
        pl.kernel, mesh=mesh,
        compiler_params=pltpu.CompilerParams(use_tc_tiling_on_sc=False),
        out_type=jax.ShapeDtypeStruct((2 * HALF, F), jnp.float32),
        scratch_types=[
            pltpu.VMEM((CH,), jnp.int32),
            pltpu.VMEM((CH,), jnp.int32),
            pltpu.VMEM((CH, F), jnp.float32),
            pltpu.VMEM_SHARED((NACC, F), jnp.float32),
            pltpu.SemaphoreType.DMA,
        ],
    )
    def k(vals_h, idx_h, zin_h, out_h, idxb, lidxb, valb, accum, sem):
        cid = lax.axis_index("c")
        sid = lax.axis_index("s")
        nbase = cid * HALF
        pltpu.sync_copy(zin_h.at[pl.ds(sid * STRIPE, STRIPE), :],
                        accum.at[pl.ds(sid * STRIPE, STRIPE), :])
        plsc.subcore_barrier()
        base = sid * per_w

        def body(ch, carry):
            off = base + ch * CH
            pltpu.sync_copy(idx_h.at[pl.ds(off, CH)], idxb)
            for j in range(CH // 16):
                iv = idxb[pl.ds(j * 16, 16)]
                local = iv - nbase
                ok = (local >= 0) & (local < HALF)
                lidxb[pl.ds(j * 16, 16)] = jnp.where(ok, local, DUMP)
            pltpu.sync_copy(vals_h.at[pl.ds(off, CH), :], valb)
            pltpu.sync_copy(valb, accum.at[lidxb], add=True)
            return carry

        lax.fori_loop(0, nch, body, 0)
        plsc.subcore_barrier()
        pltpu.sync_copy(accum.at[pl.ds(sid * 3125, 3125), :],
                        out_h.at[pl.ds(nbase + sid * 3125, 3125), :])

    return k(vals, idx, zinit)


def kernel(atomic_numbers, positions, dst_idx, src_idx, batch_segments,
           batch_size, batch_mask, atom_mask, emb, Wb, W1, b1, W2, b2, W3, b3,
           We1, We2, element_bias):
    n = positions.shape[0]
    e = dst_idx.shape[0]
    r = e // 4

    z = atomic_numbers.astype(jnp.int32).reshape(n, 1)
    src = src_idx.astype(jnp.int32)
    dst = dst_idx.astype(jnp.int32)
    segs = batch_segments.astype(jnp.int32).reshape(n, 1)
    am = atom_mask.reshape(n, 1)
    pos32 = jnp.pad(positions, ((0, 0), (0, F - 3)))
    embp = jnp.pad(emb, ((0, NZP - emb.shape[0]), (0, 0)))
    ebp = jnp.pad(element_bias.reshape(-1, 1),
                  ((0, NZP - element_bias.shape[0]), (0, 0)))
    w1t = jnp.swapaxes(W1, -1, -2)
    w2t = jnp.swapaxes(W2, -1, -2)
    w3t = jnp.swapaxes(W3, -1, -2)
    we1t = We1.reshape(1, F)
    b3r = b3.reshape(NITER, 1, F)
    binom128 = jnp.tile(jnp.concatenate([jnp.asarray(_BINOM),
                                         jnp.zeros((16,), jnp.float32)]), 4)
    binom128 = binom128.reshape(1, 128)
    eye4 = jnp.eye(4, dtype=jnp.float32)
    bdones = jnp.kron(eye4, jnp.ones((F, F), jnp.float32))
    wbpad = jnp.pad(Wb, ((0, 0), (0, F - NB), (0, 0)))
    wbtpad = jnp.pad(jnp.swapaxes(Wb, -1, -2), ((0, 0), (0, 0), (0, F - NB)))
    bdwb = [jnp.kron(eye4, wbpad[i]) for i in range(NITER)]
    bdwbt = [jnp.kron(eye4, wbtpad[i]) for i in range(NITER)]
    zinit = jnp.zeros((NACC, F), jnp.float32)

    psrc = sc_gather(pos32, src).reshape(r, 128)
    pdst = sc_gather(pos32, dst).reshape(r, 128)
    disp_p, basis_p = tc_basis(psrc, pdst, bdones, binom128)
    x0, eb = tc_embed(z, embp, ebp)

    xs, us, xsrcs = [x0], [], []
    x = x0
    for i in range(NITER):
        xsrc_p = sc_gather(x, src).reshape(r, 128)
        xsrcs.append(xsrc_p)
        msg_p = tc_msg(xsrc_p, basis_p, bdwb[i])
        u = sc_scatter_add(msg_p.reshape(e, F), dst, zinit)
        us.append(u)
        x = tc_mlp(u, W1[i], b1[i], W2[i], b2[i], W3[i], b3r[i])
        xs.append(x)

    ae, g_x = tc_head(x, eb, am, We1, we1t, We2)
    energy = tc_energy(ae, segs)[:1000, 0]

    gbs = [None, None]
    for i in range(NITER - 1, -1, -1):
        g_u = tc_mlp_bwd(us[i], g_x, W1[i], b1[i], W2[i], b2[i], W3[i], b3r[i],
                         w1t[i], w2t[i], w3t[i])
        gudst_p = sc_gather(g_u, dst).reshape(r, 128)
        if i > 0:
            gbs[i], gscat_p = tc_edge_bwd(gudst_p, xsrcs[i], basis_p,
                                          bdwb[i], bdwbt[i], True)
            g_x = sc_scatter_add(gscat_p.reshape(e, F), src, zinit)
        else:
            (gbs[i],) = tc_edge_bwd(gudst_p, xsrcs[i], basis_p,
                                    bdwb[i], bdwbt[i], False)

    gdisp_p = tc_geom_bwd(disp_p, gbs[0], gbs[1], bdones, binom128)
    gdisp = gdisp_p.reshape(e, F)
    ssrc = sc_scatter_add(gdisp, src, zinit)
    sdst = sc_scatter_add(gdisp, dst, zinit)
    forces = tc_finalize(ssrc, sdst, am)[:, :3]
    return energy, forces


# speedup vs baseline: 4.6597x; 1.4349x over previous
import functools
import math

import jax
import jax.numpy as jnp
import numpy as np
from jax import lax
from jax.experimental import pallas as pl
from jax.experimental.pallas import tpu as pltpu
from jax.experimental.pallas import tpu_sc as plsc

NB = 16
CUTOFF = 6.0
NITER = 2
NRES = 3
F = 32
NZP = 128

_BINOM = np.array([math.comb(NB - 1, k) for k in range(NB)], dtype=np.float32)


def _silu(x):
    return x * jax.nn.sigmoid(x)


def _dsilu(x):
    s = jax.nn.sigmoid(x)
    return s * (1.0 + x * (1.0 - s))


def _basis_math(d, binom128):
    u = d / (1.0 + d)
    du_dd = 1.0 / ((1.0 + d) * (1.0 + d))
    ks = jnp.remainder(
        lax.broadcasted_iota(jnp.int32, (1, 128), 1), 32).astype(jnp.float32)
    n = float(NB - 1)
    logu = jnp.log(u)
    log1mu = jnp.log(1.0 - u)
    up = jnp.exp(ks * logu)
    um = jnp.exp((n - ks) * log1mu)
    bern = binom128 * up * um
    upm1 = jnp.exp(jnp.maximum(ks - 1.0, 0.0) * logu)
    umm1 = jnp.exp(jnp.maximum(n - ks - 1.0, 0.0) * log1mu)
    dbern_du = binom128 * (ks * upm1 * um - (n - ks) * up * umm1)
    x = d / CUTOFF
    inb = x < 1.0
    den = jnp.where(inb, 1.0 - x * x, 1.0)
    cut = jnp.where(inb, jnp.exp(1.0 - 1.0 / den), 0.0)
    dcut_dd = jnp.where(inb, cut * (-2.0 * x / (den * den)) * (1.0 / CUTOFF), 0.0)
    basis = bern * cut
    dbasis = dbern_du * (du_dd * cut) + bern * dcut_dd
    return basis, dbasis


EBLK = 1000
NBLK = 2000


def _full_spec(shape):
    nd = len(shape)
    return pl.BlockSpec(shape, lambda i: (0,) * nd)


def _row_spec(blk, ncol):
    return pl.BlockSpec((blk, ncol), lambda i: (i, 0))


def _tc_call(body, grid, in_specs, out_specs, out_shape):
    return pl.pallas_call(body, grid=(grid,), in_specs=in_specs,
                          out_specs=out_specs, out_shape=out_shape)


def _embed_body(z_ref, embp_ref, ebp_ref, x0_ref, eb_ref):
    z = z_ref[...]
    cols = lax.broadcasted_iota(jnp.int32, (z.shape[0], NZP), 1)
    onehot = jnp.where(cols == z, 1.0, 0.0).astype(jnp.float32)
    x0_ref[...] = jnp.dot(onehot, embp_ref[...], preferred_element_type=jnp.float32,
                          precision=lax.Precision.HIGHEST)
    eb_ref[...] = jnp.dot(onehot, ebp_ref[...], preferred_element_type=jnp.float32,
                          precision=lax.Precision.HIGHEST)


def tc_embed(z, embp, ebp):
    n = z.shape[0]
    return _tc_call(
        _embed_body, n // NBLK,
        [_row_spec(NBLK, 1), _full_spec((NZP, F)), _full_spec((NZP, 1))],
        [_row_spec(NBLK, F), _row_spec(NBLK, 1)],
        [jax.ShapeDtypeStruct((n, F), jnp.float32),
         jax.ShapeDtypeStruct((n, 1), jnp.float32)],
    )(z, embp, ebp)


def _basis_body(psrc_ref, pdst_ref, bdones_ref, binom_ref, disp_ref, basis_ref):
    disp = psrc_ref[...] - pdst_ref[...]
    disp_ref[...] = disp
    d2 = jnp.dot(disp * disp, bdones_ref[...], preferred_element_type=jnp.float32,
                 precision=lax.Precision.HIGHEST) + 1e-12
    d = jnp.sqrt(d2)
    basis, _ = _basis_math(d, binom_ref[...])
    basis_ref[...] = basis


def tc_basis(psrc_p, pdst_p, bdones, binom128):
    r = psrc_p.shape[0]
    return _tc_call(
        _basis_body, r // EBLK,
        [_row_spec(EBLK, 128), _row_spec(EBLK, 128),
         _full_spec((128, 128)), _full_spec((1, 128))],
        [_row_spec(EBLK, 128), _row_spec(EBLK, 128)],
        [jax.ShapeDtypeStruct((r, 128), jnp.float32),
         jax.ShapeDtypeStruct((r, 128), jnp.float32)],
    )(psrc_p, pdst_p, bdones, binom128)


def _msg_body(xsrc_ref, basis_ref, bdwb_ref, msg_ref):
    radial = jnp.dot(basis_ref[...], bdwb_ref[...],
                     preferred_element_type=jnp.float32)
    msg_ref[...] = xsrc_ref[...] * radial


def tc_msg(xsrc_p, basis_p, bdwb):
    r = xsrc_p.shape[0]
    return _tc_call(
        _msg_body, r // EBLK,
        [_row_spec(EBLK, 128), _row_spec(EBLK, 128), _full_spec((128, 128))],
        _row_spec(EBLK, 128),
        jax.ShapeDtypeStruct((r, 128), jnp.float32),
    )(xsrc_p, basis_p, bdwb)


def _mlp_fwd_math(a, w1, b1, w2, b2, w3, b3):
    As, Hs = [], []
    q = a
    for r in range(NRES):
        As.append(a)
        s = _silu(a)
        h = jnp.dot(s, w1[r], preferred_element_type=jnp.float32) + b1[r]
        Hs.append(h)
        t = jnp.maximum(h, 0.0)
        q = jnp.dot(t, w2[r], preferred_element_type=jnp.float32) + b2[r]
        a = a + q
    v = jnp.dot(q, w3, preferred_element_type=jnp.float32) + b3
    return a + _silu(v), (As, Hs, v)


def _mlp_body(u_ref, w1_ref, b1_ref, w2_ref, b2_ref, w3_ref, b3_ref, x_ref):
    b1 = b1_ref[...]
    b2 = b2_ref[...]
    x, _ = _mlp_fwd_math(
        u_ref[...], w1_ref[...], [b1[r:r + 1] for r in range(NRES)],
        w2_ref[...], [b2[r:r + 1] for r in range(NRES)],
        w3_ref[...], b3_ref[...])
    x_ref[...] = x


def tc_mlp(u, w1, b1, w2, b2, w3, b3):
    n = u.shape[0]
    return _tc_call(
        _mlp_body, n // NBLK,
        [_row_spec(NBLK, F), _full_spec((NRES, F, F)), _full_spec((NRES, F)),
         _full_spec((NRES, F, F)), _full_spec((NRES, F)),
         _full_spec((F, F)), _full_spec((1, F))],
        _row_spec(NBLK, F),
        jax.ShapeDtypeStruct((n, F), jnp.float32),
    )(u, w1, b1, w2, b2, w3, b3)


def _mlp_bwd_body(u_ref, g_ref, w1_ref, b1_ref, w2_ref, b2_ref, w3_ref, b3_ref,
                  w1t_ref, w2t_ref, w3t_ref, gu_ref):
    b1 = b1_ref[...]
    b2 = b2_ref[...]
    _, (As, Hs, v) = _mlp_fwd_math(
        u_ref[...], w1_ref[...], [b1[r:r + 1] for r in range(NRES)],
        w2_ref[...], [b2[r:r + 1] for r in range(NRES)],
        w3_ref[...], b3_ref[...])
    g_out = g_ref[...]
    g_a = g_out
    g_q_extra = jnp.dot(g_out * _dsilu(v), w3t_ref[...],
                        preferred_element_type=jnp.float32)
    w1t = w1t_ref[...]
    w2t = w2t_ref[...]
    for r in range(NRES - 1, -1, -1):
        g_q = g_a + g_q_extra if r == NRES - 1 else g_a
        g_t = jnp.dot(g_q, w2t[r], preferred_element_type=jnp.float32)
        g_h = jnp.where(Hs[r] > 0.0, g_t, 0.0)
        g_s = jnp.dot(g_h, w1t[r], preferred_element_type=jnp.float32)
        g_a = g_a + g_s * _dsilu(As[r])
    gu_ref[...] = g_a


def tc_mlp_bwd(u, g_out, w1, b1, w2, b2, w3, b3, w1t, w2t, w3t):
    n = u.shape[0]
    return _tc_call(
        _mlp_bwd_body, n // NBLK,
        [_row_spec(NBLK, F), _row_spec(NBLK, F),
         _full_spec((NRES, F, F)), _full_spec((NRES, F)),
         _full_spec((NRES, F, F)), _full_spec((NRES, F)),
         _full_spec((F, F)), _full_spec((1, F)),
         _full_spec((NRES, F, F)), _full_spec((NRES, F, F)), _full_spec((F, F))],
        _row_spec(NBLK, F),
        jax.ShapeDtypeStruct((n, F), jnp.float32),
    )(u, g_out, w1, b1, w2, b2, w3, b3, w1t, w2t, w3t)


def _head_body(x2_ref, eb_ref, am_ref, we1_ref, we1t_ref, we2_ref,
               ae_ref, gx2_ref):
    x2 = x2_ref[...]
    am = am_ref[...]
    t = jnp.dot(x2, we1_ref[...], preferred_element_type=jnp.float32)
    we2 = we2_ref[...]
    ae_ref[...] = (_silu(t) * we2 + eb_ref[...]) * am
    g_t = (-am) * we2 * _dsilu(t)
    gx2_ref[...] = g_t * we1t_ref[...]


def tc_head(x2, eb, am, we1, we1t, we2):
    n = x2.shape[0]
    return _tc_call(
        _head_body, n // NBLK,
        [_row_spec(NBLK, F), _row_spec(NBLK, 1), _row_spec(NBLK, 1),
         _full_spec((F, 1)), _full_spec((1, F)), _full_spec((1, 1))],
        [_row_spec(NBLK, 1), _row_spec(NBLK, F)],
        [jax.ShapeDtypeStruct((n, 1), jnp.float32),
         jax.ShapeDtypeStruct((n, F), jnp.float32)],
    )(x2, eb, am, we1, we1t, we2)


def _energy_body(ae_ref, segs_ref, out_ref):
    @pl.when(pl.program_id(0) == 0)
    def _():
        out_ref[...] = jnp.zeros_like(out_ref)
    segs = segs_ref[...]
    cols = lax.broadcasted_iota(jnp.int32, (segs.shape[0], 1024), 1)
    onehot = jnp.where(cols == segs, 1.0, 0.0).astype(jnp.float32)
    out_ref[...] += lax.dot_general(
        onehot, ae_ref[...], (((0,), (0,)), ((), ())),
        preferred_element_type=jnp.float32, precision=lax.Precision.HIGHEST)


def tc_energy(ae, segs):
    n = ae.shape[0]
    return _tc_call(
        _energy_body, n // NBLK,
        [_row_spec(NBLK, 1), _row_spec(NBLK, 1)],
        _full_spec((1024, 1)),
        jax.ShapeDtypeStruct((1024, 1), jnp.float32),
    )(ae, segs)


def _edge_bwd_body(with_gscat, gudst_ref, xsrc_ref, basis_ref, bdwb_ref,
                   bdwbt_ref, gbasis_ref, *rest):
    gudst = gudst_ref[...]
    gbasis_ref[...] = jnp.dot(gudst * xsrc_ref[...], bdwbt_ref[...],
                              preferred_element_type=jnp.float32)
    if with_gscat:
        radial = jnp.dot(basis_ref[...], bdwb_ref[...],
                         preferred_element_type=jnp.float32)
        rest[0][...] = gudst * radial


def tc_edge_bwd(gudst_p, xsrc_p, basis_p, bdwb, bdwbt, with_gscat):
    r = gudst_p.shape[0]
    out_specs = [_row_spec(EBLK, 128)]
    out_shape = [jax.ShapeDtypeStruct((r, 128), jnp.float32)]
    if with_gscat:
        out_specs.append(_row_spec(EBLK, 128))
        out_shape.append(jax.ShapeDtypeStruct((r, 128), jnp.float32))
    return _tc_call(
        functools.partial(_edge_bwd_body, with_gscat), r // EBLK,
        [_row_spec(EBLK, 128), _row_spec(EBLK, 128), _row_spec(EBLK, 128),
         _full_spec((128, 128)), _full_spec((128, 128))],
        out_specs, out_shape,
    )(gudst_p, xsrc_p, basis_p, bdwb, bdwbt)


def _geom_bwd_body(disp_ref, gb0_ref, gb1_ref, bdones_ref, binom_ref, gdisp_ref):
    disp = disp_ref[...]
    d2 = jnp.dot(disp * disp, bdones_ref[...], preferred_element_type=jnp.float32,
                 precision=lax.Precision.HIGHEST) + 1e-12
    d = jnp.sqrt(d2)
    _, dbasis = _basis_math(d, binom_ref[...])
    gb = gb0_ref[...] + gb1_ref[...]
    g_d = jnp.dot(gb * dbasis, bdones_ref[...], preferred_element_type=jnp.float32,
                  precision=lax.Precision.HIGHEST)
    gdisp_ref[...] = (g_d / d) * disp


def tc_geom_bwd(disp_p, gb0_p, gb1_p, bdones, binom128):
    r = disp_p.shape[0]
    return _tc_call(
        _geom_bwd_body, r // EBLK,
        [_row_spec(EBLK, 128), _row_spec(EBLK, 128), _row_spec(EBLK, 128),
         _full_spec((128, 128)), _full_spec((1, 128))],
        _row_spec(EBLK, 128),
        jax.ShapeDtypeStruct((r, 128), jnp.float32),
    )(disp_p, gb0_p, gb1_p, bdones, binom128)


def _finalize_body(ssrc_ref, sdst_ref, am_ref, f_ref):
    f_ref[...] = (ssrc_ref[...] - sdst_ref[...]) * am_ref[...]


def tc_finalize(ssrc, sdst, am):
    n = ssrc.shape[0]
    return _tc_call(
        _finalize_body, n // NBLK,
        [_row_spec(NBLK, F), _row_spec(NBLK, F), _row_spec(NBLK, 1)],
        _row_spec(NBLK, F),
        jax.ShapeDtypeStruct((n, F), jnp.float32),
    )(ssrc, sdst, am)


NCORES = 2
NSUB = 16
NW = NCORES * NSUB
CH = 80
IB = 25
SIB = 5
HALF = 50000
STRIPE = 3128
NACC = NSUB * STRIPE
DUMP = 50040


def sc_gather(table, idx):
    e = idx.shape[0]
    per_w = e // NW
    nch = per_w // CH
    mesh = plsc.VectorSubcoreMesh(core_axis_name="c", subcore_axis_name="s")

    @functools.partial(
        pl.kernel, mesh=mesh,
        compiler_params=pltpu.CompilerParams(use_tc_tiling_on_sc=False),
        out_type=jax.ShapeDtypeStruct((e, F), jnp.float32),
        scratch_types=[
            pltpu.VMEM((IB * CH,), jnp.int32),
            pltpu.VMEM((IB * CH, F), jnp.float32),
            pltpu.SemaphoreType.DMA,
        ],
    )
    def k(table_h, idx_h, out_h, idxb, rowb, sem):
        wid = lax.axis_index("s") * NCORES + lax.axis_index("c")
        base = wid * per_w

        def group(g, carry):
            off = base + g * (IB * CH)
            pltpu.sync_copy(idx_h.at[pl.ds(off, IB * CH)], idxb)
            descs = []
            for j in range(IB):
                descs.append(pltpu.async_copy(
                    table_h.at[idxb.at[pl.ds(j * CH, CH)]],
                    rowb.at[pl.ds(j * CH, CH), :], sem))
            for dsc in descs:
                dsc.wait()
            pltpu.sync_copy(rowb, out_h.at[pl.ds(off, IB * CH), :])
            return carry

        lax.fori_loop(0, nch // IB, group, 0)

    return k(table, idx)


def sc_scatter_add(vals, idx, zinit):
    e = idx.shape[0]
    per_w = e // NSUB
    nch = per_w // CH
    mesh = plsc.VectorSubcoreMesh(core_axis_name="c", subcore_axis_name="s")

    @functools.partial(
        pl.kernel, mesh=mesh,
        compiler_params=pltpu.CompilerParams(use_tc_tiling_on_sc=False),
        out_type=jax.ShapeDtypeStruct((2 * HALF, F), jnp.float32),
        scratch_types=[
            pltpu.VMEM((SIB * CH,), jnp.int32),
            pltpu.VMEM((SIB, CH), jnp.int32),
            pltpu.VMEM((SIB * CH, F), jnp.float32),
            pltpu.VMEM_SHARED((NACC, F), jnp.float32),
            pltpu.SemaphoreType.DMA,
            pltpu.SemaphoreType.DMA,
        ],
    )
    def k(vals_h, idx_h, zin_h, out_h, idxb, lidxb, valb, accum, semv, sems):
        cid = lax.axis_index("c")
        sid = lax.axis_index("s")
        nbase = cid * HALF
        pltpu.sync_copy(zin_h.at[pl.ds(sid * STRIPE, STRIPE), :],
                        accum.at[pl.ds(sid * STRIPE, STRIPE), :])
        plsc.subcore_barrier()
        base = sid * per_w

        def group(g, carry):
            off = base + g * (SIB * CH)
            vdesc = pltpu.async_copy(vals_h.at[pl.ds(off, SIB * CH), :], valb,
                                     semv)
            pltpu.sync_copy(idx_h.at[pl.ds(off, SIB * CH)], idxb)
            for j in range(SIB):
                for t in range(CH // 16):
                    iv = idxb[pl.ds(j * CH + t * 16, 16)]
                    local = iv - nbase
                    ok = (local >= 0) & (local < HALF)
                    lidxb[j, pl.ds(t * 16, 16)] = jnp.where(ok, local, DUMP)
            vdesc.wait()
            descs = []
            for j in range(SIB):
                descs.append(pltpu.async_copy(
                    valb.at[pl.ds(j * CH, CH), :], accum.at[lidxb.at[j]],
                    sems, add=True))
            for dsc in descs:
                dsc.wait()
            return carry

        lax.fori_loop(0, nch // SIB, group, 0)
        plsc.subcore_barrier()
        pltpu.sync_copy(accum.at[pl.ds(sid * 3125, 3125), :],
                        out_h.at[pl.ds(nbase + sid * 3125, 3125), :])

    return k(vals, idx, zinit)


def kernel(atomic_numbers, positions, dst_idx, src_idx, batch_segments,
           batch_size, batch_mask, atom_mask, emb, Wb, W1, b1, W2, b2, W3, b3,
           We1, We2, element_bias):
    n = positions.shape[0]
    e = dst_idx.shape[0]
    r = e // 4

    z = atomic_numbers.astype(jnp.int32).reshape(n, 1)
    src = src_idx.astype(jnp.int32)
    dst = dst_idx.astype(jnp.int32)
    segs = batch_segments.astype(jnp.int32).reshape(n, 1)
    am = atom_mask.reshape(n, 1)
    pos32 = jnp.pad(positions, ((0, 0), (0, F - 3)))
    embp = jnp.pad(emb, ((0, NZP - emb.shape[0]), (0, 0)))
    ebp = jnp.pad(element_bias.reshape(-1, 1),
                  ((0, NZP - element_bias.shape[0]), (0, 0)))
    w1t = jnp.swapaxes(W1, -1, -2)
    w2t = jnp.swapaxes(W2, -1, -2)
    w3t = jnp.swapaxes(W3, -1, -2)
    we1t = We1.reshape(1, F)
    b3r = b3.reshape(NITER, 1, F)
    binom128 = jnp.tile(jnp.concatenate([jnp.asarray(_BINOM),
                                         jnp.zeros((16,), jnp.float32)]), 4)
    binom128 = binom128.reshape(1, 128)
    eye4 = jnp.eye(4, dtype=jnp.float32)
    bdones = jnp.kron(eye4, jnp.ones((F, F), jnp.float32))
    wbpad = jnp.pad(Wb, ((0, 0), (0, F - NB), (0, 0)))
    wbtpad = jnp.pad(jnp.swapaxes(Wb, -1, -2), ((0, 0), (0, 0), (0, F - NB)))
    bdwb = [jnp.kron(eye4, wbpad[i]) for i in range(NITER)]
    bdwbt = [jnp.kron(eye4, wbtpad[i]) for i in range(NITER)]
    zinit = jnp.zeros((NACC, F), jnp.float32)

    psrc = sc_gather(pos32, src).reshape(r, 128)
    pdst = sc_gather(pos32, dst).reshape(r, 128)
    disp_p, basis_p = tc_basis(psrc, pdst, bdones, binom128)
    x0, eb = tc_embed(z, embp, ebp)

    xs, us, xsrcs = [x0], [], []
    x = x0
    for i in range(NITER):
        xsrc_p = sc_gather(x, src).reshape(r, 128)
        xsrcs.append(xsrc_p)
        msg_p = tc_msg(xsrc_p, basis_p, bdwb[i])
        u = sc_scatter_add(msg_p.reshape(e, F), dst, zinit)
        us.append(u)
        x = tc_mlp(u, W1[i], b1[i], W2[i], b2[i], W3[i], b3r[i])
        xs.append(x)

    ae, g_x = tc_head(x, eb, am, We1, we1t, We2)
    energy = tc_energy(ae, segs)[:1000, 0]

    gbs = [None, None]
    for i in range(NITER - 1, -1, -1):
        g_u = tc_mlp_bwd(us[i], g_x, W1[i], b1[i], W2[i], b2[i], W3[i], b3r[i],
                         w1t[i], w2t[i], w3t[i])
        gudst_p = sc_gather(g_u, dst).reshape(r, 128)
        if i > 0:
            gbs[i], gscat_p = tc_edge_bwd(gudst_p, xsrcs[i], basis_p,
                                          bdwb[i], bdwbt[i], True)
            g_x = sc_scatter_add(gscat_p.reshape(e, F), src, zinit)
        else:
            (gbs[i],) = tc_edge_bwd(gudst_p, xsrcs[i], basis_p,
                                    bdwb[i], bdwbt[i], False)

    gdisp_p = tc_geom_bwd(disp_p, gbs[0], gbs[1], bdones, binom128)
    gdisp = gdisp_p.reshape(e, F)
    ssrc = sc_scatter_add(gdisp, src, zinit)
    sdst = sc_scatter_add(gdisp, dst, zinit)
    forces = tc_finalize(ssrc, sdst, am)[:, :3]
    return energy, forces


# speedup vs baseline: 6.7557x; 1.4498x over previous
import functools
import math

import jax
import jax.numpy as jnp
import numpy as np
from jax import lax
from jax.experimental import pallas as pl
from jax.experimental.pallas import tpu as pltpu
from jax.experimental.pallas import tpu_sc as plsc

NB = 16
CUTOFF = 6.0
NITER = 2
NRES = 3
F = 32
NZP = 128

_BINOM = np.array([math.comb(NB - 1, k) for k in range(NB)], dtype=np.float32)


def _silu(x):
    return x * jax.nn.sigmoid(x)


def _dsilu(x):
    s = jax.nn.sigmoid(x)
    return s * (1.0 + x * (1.0 - s))


def _basis_math(d, binom128):
    u = d / (1.0 + d)
    du_dd = 1.0 / ((1.0 + d) * (1.0 + d))
    ks = jnp.remainder(
        lax.broadcasted_iota(jnp.int32, (1, 128), 1), 32).astype(jnp.float32)
    n = float(NB - 1)
    logu = jnp.log(u)
    log1mu = jnp.log(1.0 - u)
    up = jnp.exp(ks * logu)
    um = jnp.exp((n - ks) * log1mu)
    bern = binom128 * up * um
    upm1 = jnp.exp(jnp.maximum(ks - 1.0, 0.0) * logu)
    umm1 = jnp.exp(jnp.maximum(n - ks - 1.0, 0.0) * log1mu)
    dbern_du = binom128 * (ks * upm1 * um - (n - ks) * up * umm1)
    x = d / CUTOFF
    inb = x < 1.0
    den = jnp.where(inb, 1.0 - x * x, 1.0)
    cut = jnp.where(inb, jnp.exp(1.0 - 1.0 / den), 0.0)
    dcut_dd = jnp.where(inb, cut * (-2.0 * x / (den * den)) * (1.0 / CUTOFF), 0.0)
    basis = bern * cut
    dbasis = dbern_du * (du_dd * cut) + bern * dcut_dd
    return basis, dbasis


EBLK = 1000
NBLK = 2000


def _full_spec(shape):
    nd = len(shape)
    return pl.BlockSpec(shape, lambda i: (0,) * nd)


def _row_spec(blk, ncol):
    return pl.BlockSpec((blk, ncol), lambda i: (i, 0))


def _tc_call(body, grid, in_specs, out_specs, out_shape):
    return pl.pallas_call(body, grid=(grid,), in_specs=in_specs,
                          out_specs=out_specs, out_shape=out_shape)


def _embed_body(z_ref, embp_ref, ebp_ref, x0_ref, eb_ref):
    z = z_ref[...]
    cols = lax.broadcasted_iota(jnp.int32, (z.shape[0], NZP), 1)
    onehot = jnp.where(cols == z, 1.0, 0.0).astype(jnp.float32)
    x0_ref[...] = jnp.dot(onehot, embp_ref[...], preferred_element_type=jnp.float32,
                          precision=lax.Precision.HIGHEST)
    eb_ref[...] = jnp.dot(onehot, ebp_ref[...], preferred_element_type=jnp.float32,
                          precision=lax.Precision.HIGHEST)


def tc_embed(z, embp, ebp):
    n = z.shape[0]
    return _tc_call(
        _embed_body, n // NBLK,
        [_row_spec(NBLK, 1), _full_spec((NZP, F)), _full_spec((NZP, 1))],
        [_row_spec(NBLK, F), _row_spec(NBLK, 1)],
        [jax.ShapeDtypeStruct((n, F), jnp.float32),
         jax.ShapeDtypeStruct((n, 1), jnp.float32)],
    )(z, embp, ebp)


def _basis_body(psrc_ref, pdst_ref, bdones_ref, binom_ref, disp_ref, basis_ref):
    disp = psrc_ref[...] - pdst_ref[...]
    disp_ref[...] = disp
    d2 = jnp.dot(disp * disp, bdones_ref[...], preferred_element_type=jnp.float32,
                 precision=lax.Precision.HIGHEST) + 1e-12
    d = jnp.sqrt(d2)
    basis, _ = _basis_math(d, binom_ref[...])
    basis_ref[...] = basis


def tc_basis(psrc_p, pdst_p, bdones, binom128):
    r = psrc_p.shape[0]
    return _tc_call(
        _basis_body, r // EBLK,
        [_row_spec(EBLK, 128), _row_spec(EBLK, 128),
         _full_spec((128, 128)), _full_spec((1, 128))],
        [_row_spec(EBLK, 128), _row_spec(EBLK, 128)],
        [jax.ShapeDtypeStruct((r, 128), jnp.float32),
         jax.ShapeDtypeStruct((r, 128), jnp.float32)],
    )(psrc_p, pdst_p, bdones, binom128)


def _msg_body(xsrc_ref, basis_ref, bdwb_ref, msg_ref):
    radial = jnp.dot(basis_ref[...], bdwb_ref[...],
                     preferred_element_type=jnp.float32)
    msg_ref[...] = xsrc_ref[...] * radial


def tc_msg(xsrc_p, basis_p, bdwb):
    r = xsrc_p.shape[0]
    return _tc_call(
        _msg_body, r // EBLK,
        [_row_spec(EBLK, 128), _row_spec(EBLK, 128), _full_spec((128, 128))],
        _row_spec(EBLK, 128),
        jax.ShapeDtypeStruct((r, 128), jnp.float32),
    )(xsrc_p, basis_p, bdwb)


def _mlp_fwd_math(a, w1, b1, w2, b2, w3, b3):
    As, Hs = [], []
    q = a
    for r in range(NRES):
        As.append(a)
        s = _silu(a)
        h = jnp.dot(s, w1[r], preferred_element_type=jnp.float32) + b1[r]
        Hs.append(h)
        t = jnp.maximum(h, 0.0)
        q = jnp.dot(t, w2[r], preferred_element_type=jnp.float32) + b2[r]
        a = a + q
    v = jnp.dot(q, w3, preferred_element_type=jnp.float32) + b3
    return a + _silu(v), (As, Hs, v)


def _mlp_body(u_ref, w1_ref, b1_ref, w2_ref, b2_ref, w3_ref, b3_ref, x_ref):
    b1 = b1_ref[...]
    b2 = b2_ref[...]
    x, _ = _mlp_fwd_math(
        u_ref[...], w1_ref[...], [b1[r:r + 1] for r in range(NRES)],
        w2_ref[...], [b2[r:r + 1] for r in range(NRES)],
        w3_ref[...], b3_ref[...])
    x_ref[...] = x


def tc_mlp(u, w1, b1, w2, b2, w3, b3):
    n = u.shape[0]
    return _tc_call(
        _mlp_body, n // NBLK,
        [_row_spec(NBLK, F), _full_spec((NRES, F, F)), _full_spec((NRES, F)),
         _full_spec((NRES, F, F)), _full_spec((NRES, F)),
         _full_spec((F, F)), _full_spec((1, F))],
        _row_spec(NBLK, F),
        jax.ShapeDtypeStruct((n, F), jnp.float32),
    )(u, w1, b1, w2, b2, w3, b3)


def _mlp_bwd_body(u_ref, g_ref, w1_ref, b1_ref, w2_ref, b2_ref, w3_ref, b3_ref,
                  w1t_ref, w2t_ref, w3t_ref, gu_ref):
    b1 = b1_ref[...]
    b2 = b2_ref[...]
    _, (As, Hs, v) = _mlp_fwd_math(
        u_ref[...], w1_ref[...], [b1[r:r + 1] for r in range(NRES)],
        w2_ref[...], [b2[r:r + 1] for r in range(NRES)],
        w3_ref[...], b3_ref[...])
    g_out = g_ref[...]
    g_a = g_out
    g_q_extra = jnp.dot(g_out * _dsilu(v), w3t_ref[...],
                        preferred_element_type=jnp.float32)
    w1t = w1t_ref[...]
    w2t = w2t_ref[...]
    for r in range(NRES - 1, -1, -1):
        g_q = g_a + g_q_extra if r == NRES - 1 else g_a
        g_t = jnp.dot(g_q, w2t[r], preferred_element_type=jnp.float32)
        g_h = jnp.where(Hs[r] > 0.0, g_t, 0.0)
        g_s = jnp.dot(g_h, w1t[r], preferred_element_type=jnp.float32)
        g_a = g_a + g_s * _dsilu(As[r])
    gu_ref[...] = g_a


def tc_mlp_bwd(u, g_out, w1, b1, w2, b2, w3, b3, w1t, w2t, w3t):
    n = u.shape[0]
    return _tc_call(
        _mlp_bwd_body, n // NBLK,
        [_row_spec(NBLK, F), _row_spec(NBLK, F),
         _full_spec((NRES, F, F)), _full_spec((NRES, F)),
         _full_spec((NRES, F, F)), _full_spec((NRES, F)),
         _full_spec((F, F)), _full_spec((1, F)),
         _full_spec((NRES, F, F)), _full_spec((NRES, F, F)), _full_spec((F, F))],
        _row_spec(NBLK, F),
        jax.ShapeDtypeStruct((n, F), jnp.float32),
    )(u, g_out, w1, b1, w2, b2, w3, b3, w1t, w2t, w3t)


def _head_body(x2_ref, eb_ref, am_ref, we1_ref, we1t_ref, we2_ref,
               ae_ref, gx2_ref):
    x2 = x2_ref[...]
    am = am_ref[...]
    t = jnp.dot(x2, we1_ref[...], preferred_element_type=jnp.float32)
    we2 = we2_ref[...]
    ae_ref[...] = (_silu(t) * we2 + eb_ref[...]) * am
    g_t = (-am) * we2 * _dsilu(t)
    gx2_ref[...] = g_t * we1t_ref[...]


def tc_head(x2, eb, am, we1, we1t, we2):
    n = x2.shape[0]
    return _tc_call(
        _head_body, n // NBLK,
        [_row_spec(NBLK, F), _row_spec(NBLK, 1), _row_spec(NBLK, 1),
         _full_spec((F, 1)), _full_spec((1, F)), _full_spec((1, 1))],
        [_row_spec(NBLK, 1), _row_spec(NBLK, F)],
        [jax.ShapeDtypeStruct((n, 1), jnp.float32),
         jax.ShapeDtypeStruct((n, F), jnp.float32)],
    )(x2, eb, am, we1, we1t, we2)


def _energy_body(ae_ref, segs_ref, out_ref):
    @pl.when(pl.program_id(0) == 0)
    def _():
        out_ref[...] = jnp.zeros_like(out_ref)
    segs = segs_ref[...]
    cols = lax.broadcasted_iota(jnp.int32, (segs.shape[0], 1024), 1)
    onehot = jnp.where(cols == segs, 1.0, 0.0).astype(jnp.float32)
    out_ref[...] += lax.dot_general(
        onehot, ae_ref[...], (((0,), (0,)), ((), ())),
        preferred_element_type=jnp.float32, precision=lax.Precision.HIGHEST)


def tc_energy(ae, segs):
    n = ae.shape[0]
    return _tc_call(
        _energy_body, n // NBLK,
        [_row_spec(NBLK, 1), _row_spec(NBLK, 1)],
        _full_spec((1024, 1)),
        jax.ShapeDtypeStruct((1024, 1), jnp.float32),
    )(ae, segs)


def _edge_bwd_body(with_gscat, gudst_ref, xsrc_ref, basis_ref, bdwb_ref,
                   bdwbt_ref, gbasis_ref, *rest):
    gudst = gudst_ref[...]
    gbasis_ref[...] = jnp.dot(gudst * xsrc_ref[...], bdwbt_ref[...],
                              preferred_element_type=jnp.float32)
    if with_gscat:
        radial = jnp.dot(basis_ref[...], bdwb_ref[...],
                         preferred_element_type=jnp.float32)
        rest[0][...] = gudst * radial


def tc_edge_bwd(gudst_p, xsrc_p, basis_p, bdwb, bdwbt, with_gscat):
    r = gudst_p.shape[0]
    out_specs = [_row_spec(EBLK, 128)]
    out_shape = [jax.ShapeDtypeStruct((r, 128), jnp.float32)]
    if with_gscat:
        out_specs.append(_row_spec(EBLK, 128))
        out_shape.append(jax.ShapeDtypeStruct((r, 128), jnp.float32))
    return _tc_call(
        functools.partial(_edge_bwd_body, with_gscat), r // EBLK,
        [_row_spec(EBLK, 128), _row_spec(EBLK, 128), _row_spec(EBLK, 128),
         _full_spec((128, 128)), _full_spec((128, 128))],
        out_specs, out_shape,
    )(gudst_p, xsrc_p, basis_p, bdwb, bdwbt)


def _geom_bwd_body(disp_ref, gb0_ref, gb1_ref, bdones_ref, binom_ref, gdisp_ref):
    disp = disp_ref[...]
    d2 = jnp.dot(disp * disp, bdones_ref[...], preferred_element_type=jnp.float32,
                 precision=lax.Precision.HIGHEST) + 1e-12
    d = jnp.sqrt(d2)
    _, dbasis = _basis_math(d, binom_ref[...])
    gb = gb0_ref[...] + gb1_ref[...]
    g_d = jnp.dot(gb * dbasis, bdones_ref[...], preferred_element_type=jnp.float32,
                  precision=lax.Precision.HIGHEST)
    gdisp_ref[...] = (g_d / d) * disp


def tc_geom_bwd(disp_p, gb0_p, gb1_p, bdones, binom128):
    r = disp_p.shape[0]
    return _tc_call(
        _geom_bwd_body, r // EBLK,
        [_row_spec(EBLK, 128), _row_spec(EBLK, 128), _row_spec(EBLK, 128),
         _full_spec((128, 128)), _full_spec((1, 128))],
        _row_spec(EBLK, 128),
        jax.ShapeDtypeStruct((r, 128), jnp.float32),
    )(disp_p, gb0_p, gb1_p, bdones, binom128)


def _finalize_body(ssrc_ref, sdst_ref, am_ref, f_ref):
    f_ref[...] = (ssrc_ref[...] - sdst_ref[...]) * am_ref[...]


def tc_finalize(ssrc, sdst, am):
    n = ssrc.shape[0]
    return _tc_call(
        _finalize_body, n // NBLK,
        [_row_spec(NBLK, F), _row_spec(NBLK, F), _row_spec(NBLK, 1)],
        _row_spec(NBLK, F),
        jax.ShapeDtypeStruct((n, F), jnp.float32),
    )(ssrc, sdst, am)


NCORES = 2
NSUB = 16
NW = NCORES * NSUB
CH = 80
IB = 25
SIB = 10
NNODE = 100000
STRIPE = 6256
NACC = NSUB * STRIPE


def sc_gather(table, idx):
    e = idx.shape[0]
    per_w = e // NW
    nch = per_w // CH
    mesh = plsc.VectorSubcoreMesh(core_axis_name="c", subcore_axis_name="s")

    @functools.partial(
        pl.kernel, mesh=mesh,
        compiler_params=pltpu.CompilerParams(use_tc_tiling_on_sc=False),
        out_type=jax.ShapeDtypeStruct((e, F), jnp.float32),
        scratch_types=[
            pltpu.VMEM((IB * CH,), jnp.int32),
            pltpu.VMEM((IB * CH, F), jnp.float32),
            pltpu.SemaphoreType.DMA,
        ],
    )
    def k(table_h, idx_h, out_h, idxb, rowb, sem):
        wid = lax.axis_index("s") * NCORES + lax.axis_index("c")
        base = wid * per_w

        def group(g, carry):
            off = base + g * (IB * CH)
            pltpu.sync_copy(idx_h.at[pl.ds(off, IB * CH)], idxb)
            descs = []
            for j in range(IB):
                descs.append(pltpu.async_copy(
                    table_h.at[idxb.at[pl.ds(j * CH, CH)]],
                    rowb.at[pl.ds(j * CH, CH), :], sem))
            for dsc in descs:
                dsc.wait()
            pltpu.sync_copy(rowb, out_h.at[pl.ds(off, IB * CH), :])
            return carry

        lax.fori_loop(0, nch // IB, group, 0)

    return k(table, idx)


def sc_scatter_add(vals, idx, zinit):
    e = idx.shape[0]
    per_w = e // NSUB
    nch = per_w // CH
    mesh = plsc.VectorSubcoreMesh(core_axis_name="c", subcore_axis_name="s")

    @functools.partial(
        pl.kernel, mesh=mesh,
        compiler_params=pltpu.CompilerParams(use_tc_tiling_on_sc=False),
        out_type=jax.ShapeDtypeStruct((NNODE, F), jnp.float32),
        scratch_types=[
            pltpu.VMEM((SIB, CH), jnp.int32),
            pltpu.VMEM((SIB * CH, 16), jnp.float32),
            pltpu.VMEM_SHARED((NACC, 16), jnp.float32),
            pltpu.SemaphoreType.DMA,
            pltpu.SemaphoreType.DMA,
        ],
    )
    def k(vals_h, idx_h, zin_h, out_h, idxb, valb, accum, semv, sems):
        cid = lax.axis_index("c")
        sid = lax.axis_index("s")
        col = cid * 16
        pltpu.sync_copy(zin_h.at[pl.ds(sid * STRIPE, STRIPE), :],
                        accum.at[pl.ds(sid * STRIPE, STRIPE), :])
        plsc.subcore_barrier()
        base = sid * per_w

        def group(g, carry):
            off = base + g * (SIB * CH)
            vdesc = pltpu.async_copy(
                vals_h.at[pl.ds(off, SIB * CH), pl.ds(col, 16)], valb, semv)
            idescs = [pltpu.async_copy(idx_h.at[pl.ds(off + j * CH, CH)],
                                       idxb.at[j], semv) for j in range(SIB)]
            vdesc.wait()
            for dsc in idescs:
                dsc.wait()
            descs = []
            for j in range(SIB):
                descs.append(pltpu.async_copy(
                    valb.at[pl.ds(j * CH, CH), :], accum.at[idxb.at[j]],
                    sems, add=True))
            for dsc in descs:
                dsc.wait()
            return carry

        lax.fori_loop(0, nch // SIB, group, 0)
        plsc.subcore_barrier()
        pltpu.sync_copy(accum.at[pl.ds(sid * 6250, 6250), :],
                        out_h.at[pl.ds(sid * 6250, 6250), pl.ds(col, 16)])

    return k(vals, idx, zinit)


def kernel(atomic_numbers, positions, dst_idx, src_idx, batch_segments,
           batch_size, batch_mask, atom_mask, emb, Wb, W1, b1, W2, b2, W3, b3,
           We1, We2, element_bias):
    n = positions.shape[0]
    e = dst_idx.shape[0]
    r = e // 4

    z = atomic_numbers.astype(jnp.int32).reshape(n, 1)
    src = src_idx.astype(jnp.int32)
    dst = dst_idx.astype(jnp.int32)
    segs = batch_segments.astype(jnp.int32).reshape(n, 1)
    am = atom_mask.reshape(n, 1)
    pos32 = jnp.pad(positions, ((0, 0), (0, F - 3)))
    embp = jnp.pad(emb, ((0, NZP - emb.shape[0]), (0, 0)))
    ebp = jnp.pad(element_bias.reshape(-1, 1),
                  ((0, NZP - element_bias.shape[0]), (0, 0)))
    w1t = jnp.swapaxes(W1, -1, -2)
    w2t = jnp.swapaxes(W2, -1, -2)
    w3t = jnp.swapaxes(W3, -1, -2)
    we1t = We1.reshape(1, F)
    b3r = b3.reshape(NITER, 1, F)
    binom128 = jnp.tile(jnp.concatenate([jnp.asarray(_BINOM),
                                         jnp.zeros((16,), jnp.float32)]), 4)
    binom128 = binom128.reshape(1, 128)
    eye4 = jnp.eye(4, dtype=jnp.float32)
    bdones = jnp.kron(eye4, jnp.ones((F, F), jnp.float32))
    wbpad = jnp.pad(Wb, ((0, 0), (0, F - NB), (0, 0)))
    wbtpad = jnp.pad(jnp.swapaxes(Wb, -1, -2), ((0, 0), (0, 0), (0, F - NB)))
    bdwb = [jnp.kron(eye4, wbpad[i]) for i in range(NITER)]
    bdwbt = [jnp.kron(eye4, wbtpad[i]) for i in range(NITER)]
    zinit = jnp.zeros((NACC, 16), jnp.float32)

    psrc = sc_gather(pos32, src).reshape(r, 128)
    pdst = sc_gather(pos32, dst).reshape(r, 128)
    disp_p, basis_p = tc_basis(psrc, pdst, bdones, binom128)
    x0, eb = tc_embed(z, embp, ebp)

    xs, us, xsrcs = [x0], [], []
    x = x0
    for i in range(NITER):
        xsrc_p = sc_gather(x, src).reshape(r, 128)
        xsrcs.append(xsrc_p)
        msg_p = tc_msg(xsrc_p, basis_p, bdwb[i])
        u = sc_scatter_add(msg_p.reshape(e, F), dst, zinit)
        us.append(u)
        x = tc_mlp(u, W1[i], b1[i], W2[i], b2[i], W3[i], b3r[i])
        xs.append(x)

    ae, g_x = tc_head(x, eb, am, We1, we1t, We2)
    energy = tc_energy(ae, segs)[:1000, 0]

    gbs = [None, None]
    for i in range(NITER - 1, -1, -1):
        g_u = tc_mlp_bwd(us[i], g_x, W1[i], b1[i], W2[i], b2[i], W3[i], b3r[i],
                         w1t[i], w2t[i], w3t[i])
        gudst_p = sc_gather(g_u, dst).reshape(r, 128)
        if i > 0:
            gbs[i], gscat_p = tc_edge_bwd(gudst_p, xsrcs[i], basis_p,
                                          bdwb[i], bdwbt[i], True)
            g_x = sc_scatter_add(gscat_p.reshape(e, F), src, zinit)
        else:
            (gbs[i],) = tc_edge_bwd(gudst_p, xsrcs[i], basis_p,
                                    bdwb[i], bdwbt[i], False)

    gdisp_p = tc_geom_bwd(disp_p, gbs[0], gbs[1], bdones, binom128)
    gdisp = gdisp_p.reshape(e, F)
    ssrc = sc_scatter_add(gdisp, src, zinit)
    sdst = sc_scatter_add(gdisp, dst, zinit)
    forces = tc_finalize(ssrc, sdst, am)[:, :3]
    return energy, forces


# speedup vs baseline: 6.9540x; 1.0293x over previous
import functools
import math

import jax
import jax.numpy as jnp
import numpy as np
from jax import lax
from jax.experimental import pallas as pl
from jax.experimental.pallas import tpu as pltpu
from jax.experimental.pallas import tpu_sc as plsc

NB = 16
CUTOFF = 6.0
NITER = 2
NRES = 3
F = 32
NZP = 128

_BINOM = np.array([math.comb(NB - 1, k) for k in range(NB)], dtype=np.float32)


def _silu(x):
    return x * jax.nn.sigmoid(x)


def _dsilu(x):
    s = jax.nn.sigmoid(x)
    return s * (1.0 + x * (1.0 - s))


def _basis_math(d, binom128):
    u = d / (1.0 + d)
    du_dd = 1.0 / ((1.0 + d) * (1.0 + d))
    ks = jnp.remainder(
        lax.broadcasted_iota(jnp.int32, (1, 128), 1), 32).astype(jnp.float32)
    n = float(NB - 1)
    logu = jnp.log(u)
    log1mu = jnp.log(1.0 - u)
    up = jnp.exp(ks * logu)
    um = jnp.exp((n - ks) * log1mu)
    bern = binom128 * up * um
    upm1 = jnp.exp(jnp.maximum(ks - 1.0, 0.0) * logu)
    umm1 = jnp.exp(jnp.maximum(n - ks - 1.0, 0.0) * log1mu)
    dbern_du = binom128 * (ks * upm1 * um - (n - ks) * up * umm1)
    x = d / CUTOFF
    inb = x < 1.0
    den = jnp.where(inb, 1.0 - x * x, 1.0)
    cut = jnp.where(inb, jnp.exp(1.0 - 1.0 / den), 0.0)
    dcut_dd = jnp.where(inb, cut * (-2.0 * x / (den * den)) * (1.0 / CUTOFF), 0.0)
    basis = bern * cut
    dbasis = dbern_du * (du_dd * cut) + bern * dcut_dd
    return basis, dbasis


EBLK = 1000
NBLK = 2000


def _full_spec(shape):
    nd = len(shape)
    return pl.BlockSpec(shape, lambda i: (0,) * nd)


def _row_spec(blk, ncol):
    return pl.BlockSpec((blk, ncol), lambda i: (i, 0))


def _tc_call(body, grid, in_specs, out_specs, out_shape):
    return pl.pallas_call(body, grid=(grid,), in_specs=in_specs,
                          out_specs=out_specs, out_shape=out_shape)


def _embed_body(z_ref, embp_ref, ebp_ref, x0_ref, eb_ref):
    z = z_ref[...]
    cols = lax.broadcasted_iota(jnp.int32, (z.shape[0], NZP), 1)
    onehot = jnp.where(cols == z, 1.0, 0.0).astype(jnp.float32)
    x0_ref[...] = jnp.dot(onehot, embp_ref[...], preferred_element_type=jnp.float32,
                          precision=lax.Precision.HIGHEST)
    eb_ref[...] = jnp.dot(onehot, ebp_ref[...], preferred_element_type=jnp.float32,
                          precision=lax.Precision.HIGHEST)


def tc_embed(z, embp, ebp):
    n = z.shape[0]
    return _tc_call(
        _embed_body, n // NBLK,
        [_row_spec(NBLK, 1), _full_spec((NZP, F)), _full_spec((NZP, 1))],
        [_row_spec(NBLK, F), _row_spec(NBLK, 1)],
        [jax.ShapeDtypeStruct((n, F), jnp.float32),
         jax.ShapeDtypeStruct((n, 1), jnp.float32)],
    )(z, embp, ebp)


def _basis_body(psrc_ref, pdst_ref, bdones_ref, binom_ref, disp_ref, basis_ref):
    disp = psrc_ref[...] - pdst_ref[...]
    disp_ref[...] = disp
    d2 = jnp.dot(disp * disp, bdones_ref[...], preferred_element_type=jnp.float32,
                 precision=lax.Precision.HIGHEST) + 1e-12
    d = jnp.sqrt(d2)
    basis, _ = _basis_math(d, binom_ref[...])
    basis_ref[...] = basis


def tc_basis(psrc_p, pdst_p, bdones, binom128):
    r = psrc_p.shape[0]
    return _tc_call(
        _basis_body, r // EBLK,
        [_row_spec(EBLK, 128), _row_spec(EBLK, 128),
         _full_spec((128, 128)), _full_spec((1, 128))],
        [_row_spec(EBLK, 128), _row_spec(EBLK, 128)],
        [jax.ShapeDtypeStruct((r, 128), jnp.float32),
         jax.ShapeDtypeStruct((r, 128), jnp.float32)],
    )(psrc_p, pdst_p, bdones, binom128)


def _msg_body(xsrc_ref, basis_ref, bdwb_ref, msg_ref):
    radial = jnp.dot(basis_ref[...], bdwb_ref[...],
                     preferred_element_type=jnp.float32)
    msg_ref[...] = xsrc_ref[...] * radial


def tc_msg(xsrc_p, basis_p, bdwb):
    r = xsrc_p.shape[0]
    return _tc_call(
        _msg_body, r // EBLK,
        [_row_spec(EBLK, 128), _row_spec(EBLK, 128), _full_spec((128, 128))],
        _row_spec(EBLK, 128),
        jax.ShapeDtypeStruct((r, 128), jnp.float32),
    )(xsrc_p, basis_p, bdwb)


def _mlp_fwd_math(a, w1, b1, w2, b2, w3, b3):
    As, Hs = [], []
    q = a
    for r in range(NRES):
        As.append(a)
        s = _silu(a)
        h = jnp.dot(s, w1[r], preferred_element_type=jnp.float32) + b1[r]
        Hs.append(h)
        t = jnp.maximum(h, 0.0)
        q = jnp.dot(t, w2[r], preferred_element_type=jnp.float32) + b2[r]
        a = a + q
    v = jnp.dot(q, w3, preferred_element_type=jnp.float32) + b3
    return a + _silu(v), (As, Hs, v)


def _mlp_body(u_ref, w1_ref, b1_ref, w2_ref, b2_ref, w3_ref, b3_ref, x_ref):
    b1 = b1_ref[...]
    b2 = b2_ref[...]
    x, _ = _mlp_fwd_math(
        u_ref[...], w1_ref[...], [b1[r:r + 1] for r in range(NRES)],
        w2_ref[...], [b2[r:r + 1] for r in range(NRES)],
        w3_ref[...], b3_ref[...])
    x_ref[...] = x


def tc_mlp(u, w1, b1, w2, b2, w3, b3):
    n = u.shape[0]
    return _tc_call(
        _mlp_body, n // NBLK,
        [_row_spec(NBLK, F), _full_spec((NRES, F, F)), _full_spec((NRES, F)),
         _full_spec((NRES, F, F)), _full_spec((NRES, F)),
         _full_spec((F, F)), _full_spec((1, F))],
        _row_spec(NBLK, F),
        jax.ShapeDtypeStruct((n, F), jnp.float32),
    )(u, w1, b1, w2, b2, w3, b3)


def _mlp_bwd_body(u_ref, g_ref, w1_ref, b1_ref, w2_ref, b2_ref, w3_ref, b3_ref,
                  w1t_ref, w2t_ref, w3t_ref, gu_ref):
    b1 = b1_ref[...]
    b2 = b2_ref[...]
    _, (As, Hs, v) = _mlp_fwd_math(
        u_ref[...], w1_ref[...], [b1[r:r + 1] for r in range(NRES)],
        w2_ref[...], [b2[r:r + 1] for r in range(NRES)],
        w3_ref[...], b3_ref[...])
    g_out = g_ref[...]
    g_a = g_out
    g_q_extra = jnp.dot(g_out * _dsilu(v), w3t_ref[...],
                        preferred_element_type=jnp.float32)
    w1t = w1t_ref[...]
    w2t = w2t_ref[...]
    for r in range(NRES - 1, -1, -1):
        g_q = g_a + g_q_extra if r == NRES - 1 else g_a
        g_t = jnp.dot(g_q, w2t[r], preferred_element_type=jnp.float32)
        g_h = jnp.where(Hs[r] > 0.0, g_t, 0.0)
        g_s = jnp.dot(g_h, w1t[r], preferred_element_type=jnp.float32)
        g_a = g_a + g_s * _dsilu(As[r])
    gu_ref[...] = g_a


def tc_mlp_bwd(u, g_out, w1, b1, w2, b2, w3, b3, w1t, w2t, w3t):
    n = u.shape[0]
    return _tc_call(
        _mlp_bwd_body, n // NBLK,
        [_row_spec(NBLK, F), _row_spec(NBLK, F),
         _full_spec((NRES, F, F)), _full_spec((NRES, F)),
         _full_spec((NRES, F, F)), _full_spec((NRES, F)),
         _full_spec((F, F)), _full_spec((1, F)),
         _full_spec((NRES, F, F)), _full_spec((NRES, F, F)), _full_spec((F, F))],
        _row_spec(NBLK, F),
        jax.ShapeDtypeStruct((n, F), jnp.float32),
    )(u, g_out, w1, b1, w2, b2, w3, b3, w1t, w2t, w3t)


def _head_body(x2_ref, eb_ref, am_ref, we1_ref, we1t_ref, we2_ref,
               ae_ref, gx2_ref):
    x2 = x2_ref[...]
    am = am_ref[...]
    t = jnp.dot(x2, we1_ref[...], preferred_element_type=jnp.float32)
    we2 = we2_ref[...]
    ae_ref[...] = (_silu(t) * we2 + eb_ref[...]) * am
    g_t = (-am) * we2 * _dsilu(t)
    gx2_ref[...] = g_t * we1t_ref[...]


def tc_head(x2, eb, am, we1, we1t, we2):
    n = x2.shape[0]
    return _tc_call(
        _head_body, n // NBLK,
        [_row_spec(NBLK, F), _row_spec(NBLK, 1), _row_spec(NBLK, 1),
         _full_spec((F, 1)), _full_spec((1, F)), _full_spec((1, 1))],
        [_row_spec(NBLK, 1), _row_spec(NBLK, F)],
        [jax.ShapeDtypeStruct((n, 1), jnp.float32),
         jax.ShapeDtypeStruct((n, F), jnp.float32)],
    )(x2, eb, am, we1, we1t, we2)


def _energy_body(ae_ref, segs_ref, out_ref):
    @pl.when(pl.program_id(0) == 0)
    def _():
        out_ref[...] = jnp.zeros_like(out_ref)
    segs = segs_ref[...]
    cols = lax.broadcasted_iota(jnp.int32, (segs.shape[0], 1024), 1)
    onehot = jnp.where(cols == segs, 1.0, 0.0).astype(jnp.float32)
    out_ref[...] += lax.dot_general(
        onehot, ae_ref[...], (((0,), (0,)), ((), ())),
        preferred_element_type=jnp.float32, precision=lax.Precision.HIGHEST)


def tc_energy(ae, segs):
    n = ae.shape[0]
    return _tc_call(
        _energy_body, n // NBLK,
        [_row_spec(NBLK, 1), _row_spec(NBLK, 1)],
        _full_spec((1024, 1)),
        jax.ShapeDtypeStruct((1024, 1), jnp.float32),
    )(ae, segs)


def _edge_bwd_body(with_gscat, gudst_ref, xsrc_ref, basis_ref, bdwb_ref,
                   bdwbt_ref, gbasis_ref, *rest):
    gudst = gudst_ref[...]
    gbasis_ref[...] = jnp.dot(gudst * xsrc_ref[...], bdwbt_ref[...],
                              preferred_element_type=jnp.float32)
    if with_gscat:
        radial = jnp.dot(basis_ref[...], bdwb_ref[...],
                         preferred_element_type=jnp.float32)
        rest[0][...] = gudst * radial


def tc_edge_bwd(gudst_p, xsrc_p, basis_p, bdwb, bdwbt, with_gscat):
    r = gudst_p.shape[0]
    out_specs = [_row_spec(EBLK, 128)]
    out_shape = [jax.ShapeDtypeStruct((r, 128), jnp.float32)]
    if with_gscat:
        out_specs.append(_row_spec(EBLK, 128))
        out_shape.append(jax.ShapeDtypeStruct((r, 128), jnp.float32))
    return _tc_call(
        functools.partial(_edge_bwd_body, with_gscat), r // EBLK,
        [_row_spec(EBLK, 128), _row_spec(EBLK, 128), _row_spec(EBLK, 128),
         _full_spec((128, 128)), _full_spec((128, 128))],
        out_specs, out_shape,
    )(gudst_p, xsrc_p, basis_p, bdwb, bdwbt)


def _geom_bwd_body(disp_ref, gb0_ref, gb1_ref, bdones_ref, binom_ref, gdisp_ref):
    disp = disp_ref[...]
    d2 = jnp.dot(disp * disp, bdones_ref[...], preferred_element_type=jnp.float32,
                 precision=lax.Precision.HIGHEST) + 1e-12
    d = jnp.sqrt(d2)
    _, dbasis = _basis_math(d, binom_ref[...])
    gb = gb0_ref[...] + gb1_ref[...]
    g_d = jnp.dot(gb * dbasis, bdones_ref[...], preferred_element_type=jnp.float32,
                  precision=lax.Precision.HIGHEST)
    gdisp_ref[...] = (g_d / d) * disp


def tc_geom_bwd(disp_p, gb0_p, gb1_p, bdones, binom128):
    r = disp_p.shape[0]
    return _tc_call(
        _geom_bwd_body, r // EBLK,
        [_row_spec(EBLK, 128), _row_spec(EBLK, 128), _row_spec(EBLK, 128),
         _full_spec((128, 128)), _full_spec((1, 128))],
        _row_spec(EBLK, 128),
        jax.ShapeDtypeStruct((r, 128), jnp.float32),
    )(disp_p, gb0_p, gb1_p, bdones, binom128)


def _finalize_body(s2_ref, am_ref, f_ref):
    s2 = s2_ref[...]
    f_ref[...] = (s2[:, :16] - s2[:, 16:]) * am_ref[...]


def tc_finalize(s2, am):
    n = s2.shape[0]
    return _tc_call(
        _finalize_body, n // NBLK,
        [_row_spec(NBLK, F), _row_spec(NBLK, 1)],
        _row_spec(NBLK, 16),
        jax.ShapeDtypeStruct((n, 16), jnp.float32),
    )(s2, am)


NCORES = 2
NSUB = 16
NW = NCORES * NSUB
CH = 80
IB = 25
SIB = 10
NNODE = 100000
STRIPE = 6256
NACC = NSUB * STRIPE


def sc_gather(table, idx):
    e = idx.shape[0]
    per_w = e // NW
    nch = per_w // CH
    mesh = plsc.VectorSubcoreMesh(core_axis_name="c", subcore_axis_name="s")

    @functools.partial(
        pl.kernel, mesh=mesh,
        compiler_params=pltpu.CompilerParams(use_tc_tiling_on_sc=False),
        out_type=jax.ShapeDtypeStruct((e, F), jnp.float32),
        scratch_types=[
            pltpu.VMEM((IB * CH,), jnp.int32),
            pltpu.VMEM((IB * CH, F), jnp.float32),
            pltpu.SemaphoreType.DMA,
        ],
    )
    def k(table_h, idx_h, out_h, idxb, rowb, sem):
        wid = lax.axis_index("s") * NCORES + lax.axis_index("c")
        base = wid * per_w

        def group(g, carry):
            off = base + g * (IB * CH)
            pltpu.sync_copy(idx_h.at[pl.ds(off, IB * CH)], idxb)
            descs = []
            for j in range(IB):
                descs.append(pltpu.async_copy(
                    table_h.at[idxb.at[pl.ds(j * CH, CH)]],
                    rowb.at[pl.ds(j * CH, CH), :], sem))
            for dsc in descs:
                dsc.wait()
            pltpu.sync_copy(rowb, out_h.at[pl.ds(off, IB * CH), :])
            return carry

        lax.fori_loop(0, nch // IB, group, 0)

    return k(table, idx)


def sc_scatter_add(vals, idx, zinit):
    e = idx.shape[0]
    per_w = e // NSUB
    nch = per_w // CH
    mesh = plsc.VectorSubcoreMesh(core_axis_name="c", subcore_axis_name="s")

    @functools.partial(
        pl.kernel, mesh=mesh,
        compiler_params=pltpu.CompilerParams(use_tc_tiling_on_sc=False),
        out_type=jax.ShapeDtypeStruct((NNODE, F), jnp.float32),
        scratch_types=[
            pltpu.VMEM((SIB, CH), jnp.int32),
            pltpu.VMEM((SIB * CH, 16), jnp.float32),
            pltpu.VMEM_SHARED((NACC, 16), jnp.float32),
            pltpu.SemaphoreType.DMA,
            pltpu.SemaphoreType.DMA,
        ],
    )
    def k(vals_h, idx_h, zin_h, out_h, idxb, valb, accum, semv, sems):
        cid = lax.axis_index("c")
        sid = lax.axis_index("s")
        col = cid * 16
        pltpu.sync_copy(zin_h.at[pl.ds(sid * STRIPE, STRIPE), :],
                        accum.at[pl.ds(sid * STRIPE, STRIPE), :])
        plsc.subcore_barrier()
        base = sid * per_w

        def group(g, carry):
            off = base + g * (SIB * CH)
            vdesc = pltpu.async_copy(
                vals_h.at[pl.ds(off, SIB * CH), pl.ds(col, 16)], valb, semv)
            idescs = [pltpu.async_copy(idx_h.at[pl.ds(off + j * CH, CH)],
                                       idxb.at[j], semv) for j in range(SIB)]
            vdesc.wait()
            for dsc in idescs:
                dsc.wait()
            descs = []
            for j in range(SIB):
                descs.append(pltpu.async_copy(
                    valb.at[pl.ds(j * CH, CH), :], accum.at[idxb.at[j]],
                    sems, add=True))
            for dsc in descs:
                dsc.wait()
            return carry

        lax.fori_loop(0, nch // SIB, group, 0)
        plsc.subcore_barrier()
        pltpu.sync_copy(accum.at[pl.ds(sid * 6250, 6250), :],
                        out_h.at[pl.ds(sid * 6250, 6250), pl.ds(col, 16)])

    return k(vals, idx, zinit)


def sc_scatter_dual(vals, idx2, zinit):
    e = idx2.shape[1]
    per_w = e // NSUB
    nch = per_w // CH
    mesh = plsc.VectorSubcoreMesh(core_axis_name="c", subcore_axis_name="s")

    @functools.partial(
        pl.kernel, mesh=mesh,
        compiler_params=pltpu.CompilerParams(use_tc_tiling_on_sc=False),
        out_type=jax.ShapeDtypeStruct((NNODE, F), jnp.float32),
        scratch_types=[
            pltpu.VMEM((SIB, CH), jnp.int32),
            pltpu.VMEM((SIB * CH, 16), jnp.float32),
            pltpu.VMEM_SHARED((NACC, 16), jnp.float32),
            pltpu.SemaphoreType.DMA,
            pltpu.SemaphoreType.DMA,
        ],
    )
    def k(vals_h, idx2_h, zin_h, out_h, idxb, valb, accum, semv, sems):
        cid = lax.axis_index("c")
        sid = lax.axis_index("s")
        col = cid * 16
        pltpu.sync_copy(zin_h.at[pl.ds(sid * STRIPE, STRIPE), :],
                        accum.at[pl.ds(sid * STRIPE, STRIPE), :])
        plsc.subcore_barrier()
        base = sid * per_w

        def group(g, carry):
            off = base + g * (SIB * CH)
            vdesc = pltpu.async_copy(
                vals_h.at[pl.ds(off, SIB * CH), pl.ds(0, 16)], valb, semv)
            idescs = [pltpu.async_copy(idx2_h.at[cid, pl.ds(off + j * CH, CH)],
                                       idxb.at[j], semv) for j in range(SIB)]
            vdesc.wait()
            for dsc in idescs:
                dsc.wait()
            descs = []
            for j in range(SIB):
                descs.append(pltpu.async_copy(
                    valb.at[pl.ds(j * CH, CH), :], accum.at[idxb.at[j]],
                    sems, add=True))
            for dsc in descs:
                dsc.wait()
            return carry

        lax.fori_loop(0, nch // SIB, group, 0)
        plsc.subcore_barrier()
        pltpu.sync_copy(accum.at[pl.ds(sid * 6250, 6250), :],
                        out_h.at[pl.ds(sid * 6250, 6250), pl.ds(col, 16)])

    return k(vals, idx2, zinit)


def kernel(atomic_numbers, positions, dst_idx, src_idx, batch_segments,
           batch_size, batch_mask, atom_mask, emb, Wb, W1, b1, W2, b2, W3, b3,
           We1, We2, element_bias):
    n = positions.shape[0]
    e = dst_idx.shape[0]
    r = e // 4

    z = atomic_numbers.astype(jnp.int32).reshape(n, 1)
    src = src_idx.astype(jnp.int32)
    dst = dst_idx.astype(jnp.int32)
    segs = batch_segments.astype(jnp.int32).reshape(n, 1)
    am = atom_mask.reshape(n, 1)
    pos32 = jnp.pad(positions, ((0, 0), (0, F - 3)))
    embp = jnp.pad(emb, ((0, NZP - emb.shape[0]), (0, 0)))
    ebp = jnp.pad(element_bias.reshape(-1, 1),
                  ((0, NZP - element_bias.shape[0]), (0, 0)))
    w1t = jnp.swapaxes(W1, -1, -2)
    w2t = jnp.swapaxes(W2, -1, -2)
    w3t = jnp.swapaxes(W3, -1, -2)
    we1t = We1.reshape(1, F)
    b3r = b3.reshape(NITER, 1, F)
    binom128 = jnp.tile(jnp.concatenate([jnp.asarray(_BINOM),
                                         jnp.zeros((16,), jnp.float32)]), 4)
    binom128 = binom128.reshape(1, 128)
    eye4 = jnp.eye(4, dtype=jnp.float32)
    bdones = jnp.kron(eye4, jnp.ones((F, F), jnp.float32))
    wbpad = jnp.pad(Wb, ((0, 0), (0, F - NB), (0, 0)))
    wbtpad = jnp.pad(jnp.swapaxes(Wb, -1, -2), ((0, 0), (0, 0), (0, F - NB)))
    bdwb = [jnp.kron(eye4, wbpad[i]) for i in range(NITER)]
    bdwbt = [jnp.kron(eye4, wbtpad[i]) for i in range(NITER)]
    zinit = jnp.zeros((NACC, 16), jnp.float32)

    psrc = sc_gather(pos32, src).reshape(r, 128)
    pdst = sc_gather(pos32, dst).reshape(r, 128)
    disp_p, basis_p = tc_basis(psrc, pdst, bdones, binom128)
    x0, eb = tc_embed(z, embp, ebp)

    xs, us, xsrcs = [x0], [], []
    x = x0
    for i in range(NITER):
        xsrc_p = sc_gather(x, src).reshape(r, 128)
        xsrcs.append(xsrc_p)
        msg_p = tc_msg(xsrc_p, basis_p, bdwb[i])
        u = sc_scatter_add(msg_p.reshape(e, F), dst, zinit)
        us.append(u)
        x = tc_mlp(u, W1[i], b1[i], W2[i], b2[i], W3[i], b3r[i])
        xs.append(x)

    ae, g_x = tc_head(x, eb, am, We1, we1t, We2)
    energy = tc_energy(ae, segs)[:1000, 0]

    gbs = [None, None]
    for i in range(NITER - 1, -1, -1):
        g_u = tc_mlp_bwd(us[i], g_x, W1[i], b1[i], W2[i], b2[i], W3[i], b3r[i],
                         w1t[i], w2t[i], w3t[i])
        gudst_p = sc_gather(g_u, dst).reshape(r, 128)
        if i > 0:
            gbs[i], gscat_p = tc_edge_bwd(gudst_p, xsrcs[i], basis_p,
                                          bdwb[i], bdwbt[i], True)
            g_x = sc_scatter_add(gscat_p.reshape(e, F), src, zinit)
        else:
            (gbs[i],) = tc_edge_bwd(gudst_p, xsrcs[i], basis_p,
                                    bdwb[i], bdwbt[i], False)

    gdisp_p = tc_geom_bwd(disp_p, gbs[0], gbs[1], bdones, binom128)
    gdisp = gdisp_p.reshape(e, F)
    idx2 = jnp.stack([src, dst])
    s2 = sc_scatter_dual(gdisp, idx2, zinit)
    forces = tc_finalize(s2, am)[:, :3]
    return energy, forces
